# Initial kernel scaffold; baseline (speedup 1.0000x reference)
import functools
import jax
import jax.numpy as jnp
from jax import lax
from jax.experimental import pallas as pl
from jax.experimental.pallas import tpu as pltpu
from jax.experimental.pallas import tpu_sc as plsc

N_NODES = 10000
N_EDGES = 320000
NBATCH = 64
D = 128
CHUNK = 128
NCHUNK = N_EDGES // CHUNK
NWORK = 32
BASE_CNT = NCHUNK // NWORK
REM = NCHUNK - BASE_CNT * NWORK
ROWS_PT = N_NODES // 16

_mesh = plsc.VectorSubcoreMesh(core_axis_name="c", subcore_axis_name="s")


def _worker_id():
    c = lax.axis_index("c")
    s = lax.axis_index("s")
    return c, s, c * 16 + s


def _segsum_body(table_hbm, src_hbm, dst_hbm, ew_hbm, out_hbm,
                 idx_v, dsti_v, ew_v, rows_v, acc_sh, sem):
    c, s, w = _worker_id()

    zv = jnp.zeros((16,), jnp.float32)

    def zrow(i, _):
        for f in range(D // 16):
            rows_v[i, pl.ds(16 * f, 16)] = zv
        return 0

    lax.fori_loop(0, CHUNK, zrow, 0)
    for k, sz in enumerate((128, 128, 128, 128, 113)):
        pltpu.sync_copy(rows_v.at[pl.ds(0, sz)],
                        acc_sh.at[pl.ds(s * ROWS_PT + k * 128, sz)])
    plsc.subcore_barrier()

    cnt = BASE_CNT + jnp.where(w < REM, 1, 0)

    def chunk_body(k, _):
        base = (w + NWORK * k) * CHUNK
        pltpu.sync_copy(src_hbm.at[pl.ds(base, CHUNK)], idx_v)
        pltpu.sync_copy(dst_hbm.at[pl.ds(base, CHUNK)], dsti_v)
        pltpu.sync_copy(ew_hbm.at[pl.ds(base, CHUNK)], ew_v)
        pltpu.async_copy(table_hbm.at[idx_v], rows_v, sem).wait()

        def scale16(i, _):
            for l in range(16):
                e = i * 16 + l
                b = plsc.load_gather(ew_v, [jnp.full((16,), e, jnp.int32)])
                for f in range(D // 16):
                    rows_v[e, pl.ds(16 * f, 16)] = (
                        rows_v[e, pl.ds(16 * f, 16)] * b)
            return 0

        lax.fori_loop(0, CHUNK // 16, scale16, 0)
        pltpu.sync_copy(rows_v, acc_sh.at[dsti_v], add=True)
        return 0

    lax.fori_loop(0, cnt, chunk_body, 0)
    plsc.subcore_barrier()
    pltpu.sync_copy(acc_sh.at[pl.ds(s * ROWS_PT, ROWS_PT)],
                    out_hbm.at[c, pl.ds(s * ROWS_PT, ROWS_PT)])


def _edge_segsum(table, src, dst, ew):
    return pl.kernel(
        _segsum_body,
        out_type=jax.ShapeDtypeStruct((2, N_NODES, D), jnp.float32),
        mesh=_mesh,
        scratch_types=[
            pltpu.VMEM((CHUNK,), jnp.int32),
            pltpu.VMEM((CHUNK,), jnp.int32),
            pltpu.VMEM((CHUNK,), jnp.float32),
            pltpu.VMEM((CHUNK, D), jnp.float32),
            pltpu.VMEM_SHARED((N_NODES, D), jnp.float32),
            pltpu.SemaphoreType.DMA,
        ],
    )(table, src, dst, ew)


def _cos_gate_body(zn_hbm, g1_hbm, src_hbm, dst_hbm, ew_hbm,
                   wmu_hbm, gatep_hbm,
                   idx_v, dsti_v, ew_v, rows_a, rows_b, wmu_v, wg_v, g1_v,
                   gacc_sh, sem_a, sem_b):
    c, s, w = _worker_id()

    def zwg(i, _):
        wg_v[pl.ds(i * 16, 16)] = jnp.zeros((16,), jnp.float32)
        return 0

    lax.fori_loop(0, CHUNK // 16, zwg, 0)
    for k, sz in enumerate((128, 128, 128, 128, 112)):
        pltpu.sync_copy(wg_v.at[pl.ds(0, sz)],
                        gacc_sh.at[pl.ds(s * 624 + k * 128, sz)])

    @pl.when(s == 0)
    def _():
        pltpu.sync_copy(wg_v.at[pl.ds(0, 16)], gacc_sh.at[pl.ds(9984, 16)])

    pltpu.sync_copy(g1_hbm, g1_v)
    plsc.subcore_barrier()

    cnt = BASE_CNT + jnp.where(w < REM, 1, 0)
    lane = lax.broadcasted_iota(jnp.int32, (16,), 0)

    def chunk_body(k, _):
        base = (w + NWORK * k) * CHUNK
        pltpu.sync_copy(src_hbm.at[pl.ds(base, CHUNK)], idx_v)
        pltpu.sync_copy(dst_hbm.at[pl.ds(base, CHUNK)], dsti_v)
        pltpu.sync_copy(ew_hbm.at[pl.ds(base, CHUNK)], ew_v)
        cp_a = pltpu.async_copy(zn_hbm.at[idx_v], rows_a, sem_a)
        cp_b = pltpu.async_copy(zn_hbm.at[dsti_v], rows_b, sem_b)

        def gate16(i, _):
            sv = idx_v[pl.ds(i * 16, 16)]
            wv = ew_v[pl.ds(i * 16, 16)]
            gv = plsc.load_gather(g1_v, [sv]) * wv
            wg_v[pl.ds(i * 16, 16)] = gv
            return 0

        lax.fori_loop(0, CHUNK // 16, gate16, 0)
        pltpu.sync_copy(wg_v, gacc_sh.at[dsti_v], add=True)

        cp_a.wait()
        cp_b.wait()

        def dot16(i, _):
            out16 = jnp.zeros((16,), jnp.float32)
            for l in range(16):
                e = i * 16 + l
                acc = rows_a[e, pl.ds(0, 16)] * rows_b[e, pl.ds(0, 16)]
                for f in range(1, D // 16):
                    acc = acc + (rows_a[e, pl.ds(16 * f, 16)] *
                                 rows_b[e, pl.ds(16 * f, 16)])
                out16 = jnp.where(lane == l, jnp.sum(acc), out16)
            wmu_v[pl.ds(i * 16, 16)] = out16
            return 0

        lax.fori_loop(0, CHUNK // 16, dot16, 0)
        pltpu.sync_copy(wmu_v, wmu_hbm.at[pl.ds(base, CHUNK)])
        return 0

    lax.fori_loop(0, cnt, chunk_body, 0)
    plsc.subcore_barrier()
    pltpu.sync_copy(gacc_sh.at[pl.ds(s * 624, 624)],
                    gatep_hbm.at[c, pl.ds(s * 624, 624)])

    @pl.when(s == 0)
    def _():
        pltpu.sync_copy(gacc_sh.at[pl.ds(9984, 16)],
                        gatep_hbm.at[c, pl.ds(9984, 16)])


def _cos_gate(zn, g1, src, dst, ew):
    return pl.kernel(
        _cos_gate_body,
        out_type=(
            jax.ShapeDtypeStruct((N_EDGES,), jnp.float32),
            jax.ShapeDtypeStruct((2, N_NODES), jnp.float32),
        ),
        mesh=_mesh,
        scratch_types=[
            pltpu.VMEM((CHUNK,), jnp.int32),
            pltpu.VMEM((CHUNK,), jnp.int32),
            pltpu.VMEM((CHUNK,), jnp.float32),
            pltpu.VMEM((CHUNK, D), jnp.float32),
            pltpu.VMEM((CHUNK, D), jnp.float32),
            pltpu.VMEM((CHUNK,), jnp.float32),
            pltpu.VMEM((CHUNK,), jnp.float32),
            pltpu.VMEM((N_NODES,), jnp.float32),
            pltpu.VMEM_SHARED((N_NODES,), jnp.float32),
            pltpu.SemaphoreType.DMA,
            pltpu.SemaphoreType.DMA,
        ],
    )(zn, g1, src, dst, ew)


RB = 2000


def _tc1_body(p_ref, x_ref, w1r_ref, b1_ref, w1o_ref, w2r_ref, b2_ref,
              w2o_ref, ws_ref, bs_ref, h2_ref, hroot_ref, zstd_ref):
    agg = p_ref[0] + p_ref[1]
    dg = lambda a, w: lax.dot_general(a, w, (((1,), (1,)), ((), ())),
                                      preferred_element_type=jnp.float32)
    h = jax.nn.relu(dg(agg, w1r_ref[...]) + b1_ref[...] +
                    dg(x_ref[...], w1o_ref[...]))
    h2_ref[...] = dg(h, w2r_ref[...])
    hroot_ref[...] = dg(h, w2o_ref[...]) + b2_ref[...]
    zstd_ref[...] = jnp.exp(jnp.tanh(dg(h, ws_ref[...]) + bs_ref[...]))


def _tc1(p, x, W1_rel, b1, W1_root, W2_rel, b2, W2_root, Ws, bs):
    nb = N_NODES // RB
    full = lambda shape: pl.BlockSpec(shape, lambda i: (0,) * len(shape))
    return pl.pallas_call(
        _tc1_body,
        grid=(nb,),
        in_specs=[
            pl.BlockSpec((2, RB, D), lambda i: (0, i, 0)),
            pl.BlockSpec((RB, D), lambda i: (i, 0)),
            full((256, 128)), full((1, 256)), full((256, 128)),
            full((128, 256)), full((1, 128)), full((128, 256)),
            full((128, 256)), full((1, 128)),
        ],
        out_specs=[
            pl.BlockSpec((RB, D), lambda i: (i, 0)),
            pl.BlockSpec((RB, D), lambda i: (i, 0)),
            pl.BlockSpec((RB, D), lambda i: (i, 0)),
        ],
        out_shape=[
            jax.ShapeDtypeStruct((N_NODES, D), jnp.float32),
            jax.ShapeDtypeStruct((N_NODES, D), jnp.float32),
            jax.ShapeDtypeStruct((N_NODES, D), jnp.float32),
        ],
    )(p, x, W1_rel, b1.reshape(1, -1), W1_root, W2_rel, b2.reshape(1, -1),
      W2_root, Ws, bs.reshape(1, -1))


def _tc2_body(p_ref, hroot_ref, wp_ref, bp_ref, wgr_ref, wgo_ref, bg_ref,
              z_ref, zn_ref, g1_ref, g2b_ref):
    z = jnp.tanh(p_ref[0] + p_ref[1] + hroot_ref[...])
    z_ref[...] = z
    na = jnp.maximum(jnp.sqrt(jnp.sum(z * z, axis=1, keepdims=True)), 1e-8)
    zn_ref[...] = z / na
    x1 = lax.dot_general(z, wp_ref[...], (((1,), (1,)), ((), ())),
                         preferred_element_type=jnp.float32) + bp_ref[...]
    g1_ref[...] = jnp.sum(x1 * wgr_ref[...], axis=1, keepdims=True)
    g2b_ref[...] = (jnp.sum(x1 * wgo_ref[...], axis=1, keepdims=True) +
                    bg_ref[...])


def _tc2(p, hroot, Wp, bp, Wg_rel, Wg_root, bg_rel):
    nb = N_NODES // RB
    full = lambda shape: pl.BlockSpec(shape, lambda i: (0,) * len(shape))
    return pl.pallas_call(
        _tc2_body,
        grid=(nb,),
        in_specs=[
            pl.BlockSpec((2, RB, D), lambda i: (0, i, 0)),
            pl.BlockSpec((RB, D), lambda i: (i, 0)),
            full((128, 128)), full((1, 128)), full((1, 128)),
            full((1, 128)), full((1, 1)),
        ],
        out_specs=[
            pl.BlockSpec((RB, D), lambda i: (i, 0)),
            pl.BlockSpec((RB, D), lambda i: (i, 0)),
            pl.BlockSpec((RB, 1), lambda i: (i, 0)),
            pl.BlockSpec((RB, 1), lambda i: (i, 0)),
        ],
        out_shape=[
            jax.ShapeDtypeStruct((N_NODES, D), jnp.float32),
            jax.ShapeDtypeStruct((N_NODES, D), jnp.float32),
            jax.ShapeDtypeStruct((N_NODES, 1), jnp.float32),
            jax.ShapeDtypeStruct((N_NODES, 1), jnp.float32),
        ],
    )(p, hroot, Wp, bp.reshape(1, -1), Wg_rel, Wg_root,
      bg_rel.reshape(1, 1))


def _tc3_body(gp_ref, g2b_ref, batch_ref, z_ref, wc_ref, bc_ref, ls_ref,
              y_ref, wstd_ref):
    ones = jnp.ones((2, 1), jnp.float32)
    gate = lax.dot_general(gp_ref[...], ones, (((0,), (0,)), ((), ())),
                           preferred_element_type=jnp.float32) + g2b_ref[...]
    mask = (batch_ref[...] ==
            lax.broadcasted_iota(jnp.int32, (N_NODES, NBATCH), 1)
            ).astype(jnp.float32)
    m = jnp.max(jnp.where(mask > 0, gate, -3e38), axis=0, keepdims=True)
    m_n = jnp.sum(mask * m, axis=1, keepdims=True)
    g = jnp.exp(gate - m_n)
    ssum = jnp.sum(mask * g, axis=0, keepdims=True)
    s_n = jnp.sum(mask * ssum, axis=1, keepdims=True)
    gsm = g / (s_n + 1e-16)
    mg = mask * gsm
    pooled = lax.dot_general(mg, z_ref[...], (((0,), (0,)), ((), ())),
                             preferred_element_type=jnp.float32)
    logits = lax.dot_general(pooled, wc_ref[...], (((1,), (1,)), ((), ())),
                             preferred_element_type=jnp.float32) + bc_ref[...]
    mx = jnp.max(logits, axis=1, keepdims=True)
    ex = jnp.exp(logits - mx)
    y_ref[...] = ex / jnp.sum(ex, axis=1, keepdims=True)
    wstd_ref[...] = jnp.exp(ls_ref[...])


def _tc3(gatep, g2b, batch2d, z, Wc, bc, log_std):
    return pl.pallas_call(
        _tc3_body,
        out_shape=[
            jax.ShapeDtypeStruct((NBATCH, 2), jnp.float32),
            jax.ShapeDtypeStruct((1, 1), jnp.float32),
        ],
    )(gatep, g2b, batch2d, z, Wc, bc.reshape(1, -1), log_std.reshape(1, 1))


def kernel(x, edge_index, edge_weight, batch, W1_rel, b1_rel, W1_root,
           W2_rel, b2_rel, W2_root, Ws, bs, Wp, bp, Wg_rel, bg_rel,
           Wg_root, Wc, bc, log_std):
    src = edge_index[0]
    dst = edge_index[1]

    p1 = _edge_segsum(x, src, dst, edge_weight)
    h2, hroot, z_std = _tc1(p1, x, W1_rel, b1_rel, W1_root, W2_rel, b2_rel,
                            W2_root, Ws, bs)
    p2 = _edge_segsum(h2, src, dst, edge_weight)
    z, zn, g1, g2b = _tc2(p2, hroot, Wp, bp, Wg_rel, Wg_root, bg_rel)
    w_mu, gatep = _cos_gate(zn, g1.reshape(-1), src, dst, edge_weight)
    y, w_std = _tc3(gatep, g2b, batch.reshape(-1, 1), z, Wc, bc, log_std)

    return (y, w_mu, w_std.reshape(1), z, z, z_std)


# speedup vs baseline: 5.2451x; 5.2451x over previous
import functools
import jax
import jax.numpy as jnp
from jax import lax
from jax.experimental import pallas as pl
from jax.experimental.pallas import tpu as pltpu
from jax.experimental.pallas import tpu_sc as plsc

N_NODES = 10000
N_EDGES = 320000
NBATCH = 64
D = 128
CHUNK = 128
NCHUNK = N_EDGES // CHUNK
NWORK = 32
BASE_CNT = NCHUNK // NWORK
REM = NCHUNK - BASE_CNT * NWORK
ROWS_PT = 624
TAIL = N_NODES - 16 * ROWS_PT
TAIL_OFF = 16 * ROWS_PT
GPAD = 10240
GSH = GPAD // 16

_mesh = plsc.VectorSubcoreMesh(core_axis_name="c", subcore_axis_name="s")
_sc_params = pltpu.CompilerParams(needs_layout_passes=False)


def _worker_id():
    c = lax.axis_index("c")
    s = lax.axis_index("s")
    return c, s, c * 16 + s


def _segsum_body(table_hbm, src_hbm, dst_hbm, ew_hbm, out_hbm,
                 idx_v, dsti_v, ew_v, rows_v, acc_sh, sem):
    c, s, w = _worker_id()

    zv = jnp.zeros((16,), jnp.float32)

    def zrow(i, _):
        for f in range(D // 16):
            rows_v[i, pl.ds(16 * f, 16)] = zv
        return 0

    lax.fori_loop(0, CHUNK, zrow, 0)
    for k, sz in enumerate((128, 128, 128, 128, 112)):
        pltpu.sync_copy(rows_v.at[pl.ds(0, sz)],
                        acc_sh.at[pl.ds(s * ROWS_PT + k * 128, sz)])

    @pl.when(s == 0)
    def _():
        pltpu.sync_copy(rows_v.at[pl.ds(0, TAIL)],
                        acc_sh.at[pl.ds(TAIL_OFF, TAIL)])

    plsc.subcore_barrier()

    cnt = BASE_CNT + jnp.where(w < REM, 1, 0)
    lane = lax.broadcasted_iota(jnp.int32, (16,), 0)

    def chunk_body(k, _):
        base = (w + NWORK * k) * CHUNK
        pltpu.sync_copy(src_hbm.at[pl.ds(base, CHUNK)], idx_v)
        pltpu.sync_copy(dst_hbm.at[pl.ds(base, CHUNK)], dsti_v)
        pltpu.sync_copy(ew_hbm.at[pl.ds(base, CHUNK)], ew_v)
        pltpu.async_copy(table_hbm.at[idx_v], rows_v, sem).wait()

        def scale16(i, _):
            ew16 = ew_v[pl.ds(i * 16, 16)]
            for l in range(16):
                e = i * 16 + l
                b = jnp.sum(jnp.where(lane == l, ew16, 0.0))
                for f in range(D // 16):
                    rows_v[e, pl.ds(16 * f, 16)] = (
                        rows_v[e, pl.ds(16 * f, 16)] * b)
            return 0

        lax.fori_loop(0, CHUNK // 16, scale16, 0)
        pltpu.sync_copy(rows_v, acc_sh.at[dsti_v], add=True)
        return 0

    lax.fori_loop(0, cnt, chunk_body, 0)
    plsc.subcore_barrier()
    pltpu.sync_copy(acc_sh.at[pl.ds(s * ROWS_PT, ROWS_PT)],
                    out_hbm.at[c, pl.ds(s * ROWS_PT, ROWS_PT)])

    @pl.when(s == 0)
    def _():
        pltpu.sync_copy(acc_sh.at[pl.ds(TAIL_OFF, TAIL)],
                        out_hbm.at[c, pl.ds(TAIL_OFF, TAIL)])


def _edge_segsum(table, src, dst, ew):
    return pl.kernel(
        _segsum_body,
        out_type=jax.ShapeDtypeStruct((2, N_NODES, D), jnp.float32),
        mesh=_mesh,
        compiler_params=_sc_params,
        scratch_types=[
            pltpu.VMEM((CHUNK,), jnp.int32),
            pltpu.VMEM((CHUNK,), jnp.int32),
            pltpu.VMEM((CHUNK,), jnp.float32),
            pltpu.VMEM((CHUNK, D), jnp.float32),
            pltpu.VMEM_SHARED((N_NODES, D), jnp.float32),
            pltpu.SemaphoreType.DMA,
        ],
    )(table, src, dst, ew)


def _cos_gate_body(zn_hbm, g1_hbm, src_hbm, dst_hbm, ew_hbm,
                   wmu_hbm, gatep_hbm,
                   idx_v, dsti_v, ew_v, rows_a, rows_b, wmu_v, wg_v, g1_v,
                   gacc_sh, sem_a, sem_b, sem_g):
    c, s, w = _worker_id()

    def zwg(i, _):
        wg_v[pl.ds(i * 16, 16)] = jnp.zeros((16,), jnp.float32)
        return 0

    lax.fori_loop(0, CHUNK // 16, zwg, 0)
    for k in range(GSH // CHUNK):
        pltpu.sync_copy(wg_v, gacc_sh.at[pl.ds(s * GSH + k * CHUNK, CHUNK)])

    plsc.subcore_barrier()

    cnt = BASE_CNT + jnp.where(w < REM, 1, 0)
    lane = lax.broadcasted_iota(jnp.int32, (16,), 0)

    def chunk_body(k, _):
        base = (w + NWORK * k) * CHUNK
        pltpu.sync_copy(src_hbm.at[pl.ds(base, CHUNK)], idx_v)
        pltpu.sync_copy(dst_hbm.at[pl.ds(base, CHUNK)], dsti_v)
        pltpu.sync_copy(ew_hbm.at[pl.ds(base, CHUNK)], ew_v)
        cp_a = pltpu.async_copy(zn_hbm.at[idx_v], rows_a, sem_a)
        cp_b = pltpu.async_copy(zn_hbm.at[dsti_v], rows_b, sem_b)
        pltpu.async_copy(g1_hbm.at[idx_v], g1_v, sem_g).wait()

        def gate16(i, _):
            wv = ew_v[pl.ds(i * 16, 16)]
            wg_v[pl.ds(i * 16, 16)] = g1_v[pl.ds(i * 16, 16)] * wv
            return 0

        lax.fori_loop(0, CHUNK // 16, gate16, 0)
        pltpu.sync_copy(wg_v, gacc_sh.at[dsti_v], add=True)

        cp_a.wait()
        cp_b.wait()

        def dot16(i, _):
            out16 = jnp.zeros((16,), jnp.float32)
            for l in range(16):
                e = i * 16 + l
                acc = rows_a[e, pl.ds(0, 16)] * rows_b[e, pl.ds(0, 16)]
                for f in range(1, D // 16):
                    acc = acc + (rows_a[e, pl.ds(16 * f, 16)] *
                                 rows_b[e, pl.ds(16 * f, 16)])
                out16 = jnp.where(lane == l, jnp.sum(acc), out16)
            wmu_v[pl.ds(i * 16, 16)] = out16
            return 0

        lax.fori_loop(0, CHUNK // 16, dot16, 0)
        pltpu.sync_copy(wmu_v, wmu_hbm.at[pl.ds(base, CHUNK)])
        return 0

    lax.fori_loop(0, cnt, chunk_body, 0)
    plsc.subcore_barrier()
    pltpu.sync_copy(gacc_sh.at[pl.ds(s * GSH, GSH)],
                    gatep_hbm.at[c, 0, pl.ds(s * GSH, GSH)])


def _cos_gate(zn, g1, src, dst, ew):
    return pl.kernel(
        _cos_gate_body,
        out_type=(
            jax.ShapeDtypeStruct((N_EDGES,), jnp.float32),
            jax.ShapeDtypeStruct((2, 1, GPAD), jnp.float32),
        ),
        mesh=_mesh,
        compiler_params=_sc_params,
        scratch_types=[
            pltpu.VMEM((CHUNK,), jnp.int32),
            pltpu.VMEM((CHUNK,), jnp.int32),
            pltpu.VMEM((CHUNK,), jnp.float32),
            pltpu.VMEM((CHUNK, D), jnp.float32),
            pltpu.VMEM((CHUNK, D), jnp.float32),
            pltpu.VMEM((CHUNK,), jnp.float32),
            pltpu.VMEM((CHUNK,), jnp.float32),
            pltpu.VMEM((CHUNK,), jnp.float32),
            pltpu.VMEM_SHARED((GPAD,), jnp.float32),
            pltpu.SemaphoreType.DMA,
            pltpu.SemaphoreType.DMA,
            pltpu.SemaphoreType.DMA,
        ],
    )(zn, g1, src, dst, ew)


RB = 2000


def _tc1_body(p_ref, x_ref, w1r_ref, b1_ref, w1o_ref, w2r_ref, b2_ref,
              w2o_ref, ws_ref, bs_ref, h2_ref, hroot_ref, zstd_ref):
    agg = p_ref[0] + p_ref[1]
    dg = lambda a, w: lax.dot_general(a, w, (((1,), (1,)), ((), ())),
                                      preferred_element_type=jnp.float32)
    h = jax.nn.relu(dg(agg, w1r_ref[...]) + b1_ref[...] +
                    dg(x_ref[...], w1o_ref[...]))
    h2_ref[...] = dg(h, w2r_ref[...])
    hroot_ref[...] = dg(h, w2o_ref[...]) + b2_ref[...]
    zstd_ref[...] = jnp.exp(jnp.tanh(dg(h, ws_ref[...]) + bs_ref[...]))


def _tc1(p, x, W1_rel, b1, W1_root, W2_rel, b2, W2_root, Ws, bs):
    nb = N_NODES // RB
    full = lambda shape: pl.BlockSpec(shape, lambda i: (0,) * len(shape))
    return pl.pallas_call(
        _tc1_body,
        grid=(nb,),
        in_specs=[
            pl.BlockSpec((2, RB, D), lambda i: (0, i, 0)),
            pl.BlockSpec((RB, D), lambda i: (i, 0)),
            full((256, 128)), full((1, 256)), full((256, 128)),
            full((128, 256)), full((1, 128)), full((128, 256)),
            full((128, 256)), full((1, 128)),
        ],
        out_specs=[
            pl.BlockSpec((RB, D), lambda i: (i, 0)),
            pl.BlockSpec((RB, D), lambda i: (i, 0)),
            pl.BlockSpec((RB, D), lambda i: (i, 0)),
        ],
        out_shape=[
            jax.ShapeDtypeStruct((N_NODES, D), jnp.float32),
            jax.ShapeDtypeStruct((N_NODES, D), jnp.float32),
            jax.ShapeDtypeStruct((N_NODES, D), jnp.float32),
        ],
    )(p, x, W1_rel, b1.reshape(1, -1), W1_root, W2_rel, b2.reshape(1, -1),
      W2_root, Ws, bs.reshape(1, -1))


def _tc2_body(p_ref, hroot_ref, wp_ref, bp_ref, wgr_ref, wgo_ref, bg_ref,
              z_ref, zn_ref, g1_ref, g2b_ref):
    z = jnp.tanh(p_ref[0] + p_ref[1] + hroot_ref[...])
    z_ref[...] = z
    na = jnp.maximum(jnp.sqrt(jnp.sum(z * z, axis=1, keepdims=True)), 1e-8)
    zn_ref[...] = z / na
    x1 = lax.dot_general(z, wp_ref[...], (((1,), (1,)), ((), ())),
                         preferred_element_type=jnp.float32) + bp_ref[...]
    g1_ref[...] = jnp.sum(x1 * wgr_ref[...], axis=1, keepdims=True)
    g2b_ref[...] = (jnp.sum(x1 * wgo_ref[...], axis=1, keepdims=True) +
                    bg_ref[...])


def _tc2(p, hroot, Wp, bp, Wg_rel, Wg_root, bg_rel):
    nb = N_NODES // RB
    full = lambda shape: pl.BlockSpec(shape, lambda i: (0,) * len(shape))
    return pl.pallas_call(
        _tc2_body,
        grid=(nb,),
        in_specs=[
            pl.BlockSpec((2, RB, D), lambda i: (0, i, 0)),
            pl.BlockSpec((RB, D), lambda i: (i, 0)),
            full((128, 128)), full((1, 128)), full((1, 128)),
            full((1, 128)), full((1, 1)),
        ],
        out_specs=[
            pl.BlockSpec((RB, D), lambda i: (i, 0)),
            pl.BlockSpec((RB, D), lambda i: (i, 0)),
            pl.BlockSpec((RB, 1), lambda i: (i, 0)),
            pl.BlockSpec((RB, 1), lambda i: (i, 0)),
        ],
        out_shape=[
            jax.ShapeDtypeStruct((N_NODES, D), jnp.float32),
            jax.ShapeDtypeStruct((N_NODES, D), jnp.float32),
            jax.ShapeDtypeStruct((N_NODES, 1), jnp.float32),
            jax.ShapeDtypeStruct((N_NODES, 1), jnp.float32),
        ],
    )(p, hroot, Wp, bp.reshape(1, -1), Wg_rel, Wg_root,
      bg_rel.reshape(1, 1))


def _tc3_body(gp_ref, g2b_ref, batch_ref, z_ref, wc_ref, bc_ref, ls_ref,
              y_ref, wstd_ref):
    ones = jnp.ones((2, 1), jnp.float32)
    gate = lax.dot_general(gp_ref[...], ones, (((0,), (0,)), ((), ())),
                           preferred_element_type=jnp.float32) + g2b_ref[...]
    mask = (batch_ref[...] ==
            lax.broadcasted_iota(jnp.int32, (N_NODES, NBATCH), 1)
            ).astype(jnp.float32)
    m = jnp.max(jnp.where(mask > 0, gate, -3e38), axis=0, keepdims=True)
    m_n = jnp.sum(mask * m, axis=1, keepdims=True)
    g = jnp.exp(gate - m_n)
    ssum = jnp.sum(mask * g, axis=0, keepdims=True)
    s_n = jnp.sum(mask * ssum, axis=1, keepdims=True)
    gsm = g / (s_n + 1e-16)
    mg = mask * gsm
    pooled = lax.dot_general(mg, z_ref[...], (((0,), (0,)), ((), ())),
                             preferred_element_type=jnp.float32)
    logits = lax.dot_general(pooled, wc_ref[...], (((1,), (1,)), ((), ())),
                             preferred_element_type=jnp.float32) + bc_ref[...]
    mx = jnp.max(logits, axis=1, keepdims=True)
    ex = jnp.exp(logits - mx)
    y_ref[...] = ex / jnp.sum(ex, axis=1, keepdims=True)
    wstd_ref[...] = jnp.exp(ls_ref[...])


def _tc3(gatep, g2b, batch2d, z, Wc, bc, log_std):
    return pl.pallas_call(
        _tc3_body,
        out_shape=[
            jax.ShapeDtypeStruct((NBATCH, 2), jnp.float32),
            jax.ShapeDtypeStruct((1, 1), jnp.float32),
        ],
    )(gatep, g2b, batch2d, z, Wc, bc.reshape(1, -1), log_std.reshape(1, 1))


def kernel(x, edge_index, edge_weight, batch, W1_rel, b1_rel, W1_root,
           W2_rel, b2_rel, W2_root, Ws, bs, Wp, bp, Wg_rel, bg_rel,
           Wg_root, Wc, bc, log_std):
    src = edge_index[0]
    dst = edge_index[1]

    p1 = _edge_segsum(x, src, dst, edge_weight)
    h2, hroot, z_std = _tc1(p1, x, W1_rel, b1_rel, W1_root, W2_rel, b2_rel,
                            W2_root, Ws, bs)
    p2 = _edge_segsum(h2, src, dst, edge_weight)
    z, zn, g1, g2b = _tc2(p2, hroot, Wp, bp, Wg_rel, Wg_root, bg_rel)
    w_mu, gatep = _cos_gate(zn, g1.reshape(-1), src, dst, edge_weight)
    gatep2 = gatep.reshape(2, GPAD)[:, :N_NODES]
    y, w_std = _tc3(gatep2, g2b, batch.reshape(-1, 1), z, Wc, bc, log_std)

    return (y, w_mu, w_std.reshape(1), z, z, z_std)


# speedup vs baseline: 6.5454x; 1.2479x over previous
import functools
import jax
import jax.numpy as jnp
from jax import lax
from jax.experimental import pallas as pl
from jax.experimental.pallas import tpu as pltpu
from jax.experimental.pallas import tpu_sc as plsc

N_NODES = 10000
N_EDGES = 320000
NBATCH = 64
D = 128
CHUNK = 128
NCHUNK = N_EDGES // CHUNK
NWORK = 32
BASE_CNT = NCHUNK // NWORK
REM = NCHUNK - BASE_CNT * NWORK
ROWS_PT = 624
TAIL = N_NODES - 16 * ROWS_PT
TAIL_OFF = 16 * ROWS_PT
GPAD = 10240
GSH = GPAD // 16

_mesh = plsc.VectorSubcoreMesh(core_axis_name="c", subcore_axis_name="s")
_sc_params = pltpu.CompilerParams(needs_layout_passes=False)


def _worker_id():
    c = lax.axis_index("c")
    s = lax.axis_index("s")
    return c, s, c * 16 + s


def _scale_rows(rows_v, ew_v, lane):
    def scale16(i, _):
        ew16 = ew_v[pl.ds(i * 16, 16)]
        for l in range(16):
            e = i * 16 + l
            b = jnp.sum(jnp.where(lane == l, ew16, 0.0))
            for f in range(D // 16):
                rows_v[e, pl.ds(16 * f, 16)] = (
                    rows_v[e, pl.ds(16 * f, 16)] * b)
        return 0

    lax.fori_loop(0, CHUNK // 16, scale16, 0)


def _segsum_body(table_hbm, eidx_hbm, ew_hbm, out_hbm,
                 ei0, ei1, ew0, ew1, rows0, rows1, acc_sh,
                 gsem0, gsem1, ssem0, ssem1):
    c, s, w = _worker_id()
    eib = (ei0, ei1)
    ewb = (ew0, ew1)
    rows = (rows0, rows1)
    gsem = (gsem0, gsem1)
    ssem = (ssem0, ssem1)

    zv = jnp.zeros((16,), jnp.float32)

    def zrow(i, _):
        for f in range(D // 16):
            rows0[i, pl.ds(16 * f, 16)] = zv
        return 0

    lax.fori_loop(0, CHUNK, zrow, 0)
    for k, sz in enumerate((128, 128, 128, 128, 112)):
        pltpu.sync_copy(rows0.at[pl.ds(0, sz)],
                        acc_sh.at[pl.ds(s * ROWS_PT + k * 128, sz)])

    @pl.when(s == 0)
    def _():
        pltpu.sync_copy(rows0.at[pl.ds(0, TAIL)],
                        acc_sh.at[pl.ds(TAIL_OFF, TAIL)])

    plsc.subcore_barrier()

    cnt = BASE_CNT + jnp.where(w < REM, 1, 0)
    lane = lax.broadcasted_iota(jnp.int32, (16,), 0)

    def g_desc(b):
        return pltpu.make_async_copy(table_hbm.at[eib[b].at[0]], rows[b],
                                     gsem[b])

    def s_desc(b):
        return pltpu.make_async_copy(rows[b], acc_sh.at[eib[b].at[1]],
                                     ssem[b])

    def load_and_gather(k, b):
        base = (w + NWORK * k) * CHUNK
        pltpu.sync_copy(eidx_hbm.at[:, pl.ds(base, CHUNK)], eib[b])
        pltpu.sync_copy(ew_hbm.at[pl.ds(base, CHUNK)], ewb[b])
        g_desc(b).start()

    load_and_gather(0, 0)

    def pair_body(k2, _):
        for b in range(2):
            k = k2 * 2 + b

            @pl.when(k < cnt)
            def _():
                g_desc(b).wait()

                @pl.when(k + 1 < cnt)
                def _():
                    @pl.when(k >= 1)
                    def _():
                        s_desc(1 - b).wait()

                    load_and_gather(k + 1, 1 - b)

                _scale_rows(rows[b], ewb[b], lane)
                pltpu.async_copy(rows[b], acc_sh.at[eib[b].at[1]], ssem[b],
                                 add=True)

        return 0

    lax.fori_loop(0, (BASE_CNT + 2) // 2, pair_body, 0)
    s_desc(0).wait()
    s_desc(1).wait()
    plsc.subcore_barrier()
    pltpu.sync_copy(acc_sh.at[pl.ds(s * ROWS_PT, ROWS_PT)],
                    out_hbm.at[c, pl.ds(s * ROWS_PT, ROWS_PT)])

    @pl.when(s == 0)
    def _():
        pltpu.sync_copy(acc_sh.at[pl.ds(TAIL_OFF, TAIL)],
                        out_hbm.at[c, pl.ds(TAIL_OFF, TAIL)])


def _edge_segsum(table, eidx, ew):
    return pl.kernel(
        _segsum_body,
        out_type=jax.ShapeDtypeStruct((2, N_NODES, D), jnp.float32),
        mesh=_mesh,
        compiler_params=_sc_params,
        scratch_types=[
            pltpu.VMEM((2, CHUNK), jnp.int32),
            pltpu.VMEM((2, CHUNK), jnp.int32),
            pltpu.VMEM((CHUNK,), jnp.float32),
            pltpu.VMEM((CHUNK,), jnp.float32),
            pltpu.VMEM((CHUNK, D), jnp.float32),
            pltpu.VMEM((CHUNK, D), jnp.float32),
            pltpu.VMEM_SHARED((N_NODES, D), jnp.float32),
            pltpu.SemaphoreType.DMA,
            pltpu.SemaphoreType.DMA,
            pltpu.SemaphoreType.DMA,
            pltpu.SemaphoreType.DMA,
        ],
    )(table, eidx, ew)


def _cos_gate_body(zn_hbm, g1_hbm, src_hbm, dst_hbm, ew_hbm,
                   wmu_hbm, gatep_hbm,
                   idx_v, dsti_v, ew_v, rows_a, rows_b, wmu_v, wg_v, g1_v,
                   gacc_sh, sem_a, sem_b, sem_g):
    c, s, w = _worker_id()

    def zwg(i, _):
        wg_v[pl.ds(i * 16, 16)] = jnp.zeros((16,), jnp.float32)
        return 0

    lax.fori_loop(0, CHUNK // 16, zwg, 0)
    for k in range(GSH // CHUNK):
        pltpu.sync_copy(wg_v, gacc_sh.at[pl.ds(s * GSH + k * CHUNK, CHUNK)])

    plsc.subcore_barrier()

    cnt = BASE_CNT + jnp.where(w < REM, 1, 0)
    lane = lax.broadcasted_iota(jnp.int32, (16,), 0)

    def chunk_body(k, _):
        base = (w + NWORK * k) * CHUNK
        pltpu.sync_copy(src_hbm.at[pl.ds(base, CHUNK)], idx_v)
        pltpu.sync_copy(dst_hbm.at[pl.ds(base, CHUNK)], dsti_v)
        pltpu.sync_copy(ew_hbm.at[pl.ds(base, CHUNK)], ew_v)
        cp_a = pltpu.async_copy(zn_hbm.at[idx_v], rows_a, sem_a)
        cp_b = pltpu.async_copy(zn_hbm.at[dsti_v], rows_b, sem_b)
        pltpu.async_copy(g1_hbm.at[idx_v], g1_v, sem_g).wait()

        def gate16(i, _):
            wv = ew_v[pl.ds(i * 16, 16)]
            wg_v[pl.ds(i * 16, 16)] = g1_v[pl.ds(i * 16, 16)] * wv
            return 0

        lax.fori_loop(0, CHUNK // 16, gate16, 0)
        pltpu.sync_copy(wg_v, gacc_sh.at[dsti_v], add=True)

        cp_a.wait()
        cp_b.wait()

        def dot16(i, _):
            out16 = jnp.zeros((16,), jnp.float32)
            for l in range(16):
                e = i * 16 + l
                acc = rows_a[e, pl.ds(0, 16)] * rows_b[e, pl.ds(0, 16)]
                for f in range(1, D // 16):
                    acc = acc + (rows_a[e, pl.ds(16 * f, 16)] *
                                 rows_b[e, pl.ds(16 * f, 16)])
                out16 = jnp.where(lane == l, jnp.sum(acc), out16)
            wmu_v[pl.ds(i * 16, 16)] = out16
            return 0

        lax.fori_loop(0, CHUNK // 16, dot16, 0)
        pltpu.sync_copy(wmu_v, wmu_hbm.at[pl.ds(base, CHUNK)])
        return 0

    lax.fori_loop(0, cnt, chunk_body, 0)
    plsc.subcore_barrier()
    pltpu.sync_copy(gacc_sh.at[pl.ds(s * GSH, GSH)],
                    gatep_hbm.at[c, 0, pl.ds(s * GSH, GSH)])


def _cos_gate(zn, g1, src, dst, ew):
    return pl.kernel(
        _cos_gate_body,
        out_type=(
            jax.ShapeDtypeStruct((N_EDGES,), jnp.float32),
            jax.ShapeDtypeStruct((2, 1, GPAD), jnp.float32),
        ),
        mesh=_mesh,
        compiler_params=_sc_params,
        scratch_types=[
            pltpu.VMEM((CHUNK,), jnp.int32),
            pltpu.VMEM((CHUNK,), jnp.int32),
            pltpu.VMEM((CHUNK,), jnp.float32),
            pltpu.VMEM((CHUNK, D), jnp.float32),
            pltpu.VMEM((CHUNK, D), jnp.float32),
            pltpu.VMEM((CHUNK,), jnp.float32),
            pltpu.VMEM((CHUNK,), jnp.float32),
            pltpu.VMEM((CHUNK,), jnp.float32),
            pltpu.VMEM_SHARED((GPAD,), jnp.float32),
            pltpu.SemaphoreType.DMA,
            pltpu.SemaphoreType.DMA,
            pltpu.SemaphoreType.DMA,
        ],
    )(zn, g1, src, dst, ew)


RB = 2000


def _tc1_body(p_ref, x_ref, w1r_ref, b1_ref, w1o_ref, w2r_ref, b2_ref,
              w2o_ref, ws_ref, bs_ref, h2_ref, hroot_ref, zstd_ref):
    agg = p_ref[0] + p_ref[1]
    dg = lambda a, w: lax.dot_general(a, w, (((1,), (1,)), ((), ())),
                                      preferred_element_type=jnp.float32)
    h = jax.nn.relu(dg(agg, w1r_ref[...]) + b1_ref[...] +
                    dg(x_ref[...], w1o_ref[...]))
    h2_ref[...] = dg(h, w2r_ref[...])
    hroot_ref[...] = dg(h, w2o_ref[...]) + b2_ref[...]
    zstd_ref[...] = jnp.exp(jnp.tanh(dg(h, ws_ref[...]) + bs_ref[...]))


def _tc1(p, x, W1_rel, b1, W1_root, W2_rel, b2, W2_root, Ws, bs):
    nb = N_NODES // RB
    full = lambda shape: pl.BlockSpec(shape, lambda i: (0,) * len(shape))
    return pl.pallas_call(
        _tc1_body,
        grid=(nb,),
        in_specs=[
            pl.BlockSpec((2, RB, D), lambda i: (0, i, 0)),
            pl.BlockSpec((RB, D), lambda i: (i, 0)),
            full((256, 128)), full((1, 256)), full((256, 128)),
            full((128, 256)), full((1, 128)), full((128, 256)),
            full((128, 256)), full((1, 128)),
        ],
        out_specs=[
            pl.BlockSpec((RB, D), lambda i: (i, 0)),
            pl.BlockSpec((RB, D), lambda i: (i, 0)),
            pl.BlockSpec((RB, D), lambda i: (i, 0)),
        ],
        out_shape=[
            jax.ShapeDtypeStruct((N_NODES, D), jnp.float32),
            jax.ShapeDtypeStruct((N_NODES, D), jnp.float32),
            jax.ShapeDtypeStruct((N_NODES, D), jnp.float32),
        ],
    )(p, x, W1_rel, b1.reshape(1, -1), W1_root, W2_rel, b2.reshape(1, -1),
      W2_root, Ws, bs.reshape(1, -1))


def _tc2_body(p_ref, hroot_ref, wp_ref, bp_ref, wgr_ref, wgo_ref, bg_ref,
              z_ref, zn_ref, g1_ref, g2b_ref):
    z = jnp.tanh(p_ref[0] + p_ref[1] + hroot_ref[...])
    z_ref[...] = z
    na = jnp.maximum(jnp.sqrt(jnp.sum(z * z, axis=1, keepdims=True)), 1e-8)
    zn_ref[...] = z / na
    x1 = lax.dot_general(z, wp_ref[...], (((1,), (1,)), ((), ())),
                         preferred_element_type=jnp.float32) + bp_ref[...]
    g1_ref[...] = jnp.sum(x1 * wgr_ref[...], axis=1, keepdims=True)
    g2b_ref[...] = (jnp.sum(x1 * wgo_ref[...], axis=1, keepdims=True) +
                    bg_ref[...])


def _tc2(p, hroot, Wp, bp, Wg_rel, Wg_root, bg_rel):
    nb = N_NODES // RB
    full = lambda shape: pl.BlockSpec(shape, lambda i: (0,) * len(shape))
    return pl.pallas_call(
        _tc2_body,
        grid=(nb,),
        in_specs=[
            pl.BlockSpec((2, RB, D), lambda i: (0, i, 0)),
            pl.BlockSpec((RB, D), lambda i: (i, 0)),
            full((128, 128)), full((1, 128)), full((1, 128)),
            full((1, 128)), full((1, 1)),
        ],
        out_specs=[
            pl.BlockSpec((RB, D), lambda i: (i, 0)),
            pl.BlockSpec((RB, D), lambda i: (i, 0)),
            pl.BlockSpec((RB, 1), lambda i: (i, 0)),
            pl.BlockSpec((RB, 1), lambda i: (i, 0)),
        ],
        out_shape=[
            jax.ShapeDtypeStruct((N_NODES, D), jnp.float32),
            jax.ShapeDtypeStruct((N_NODES, D), jnp.float32),
            jax.ShapeDtypeStruct((N_NODES, 1), jnp.float32),
            jax.ShapeDtypeStruct((N_NODES, 1), jnp.float32),
        ],
    )(p, hroot, Wp, bp.reshape(1, -1), Wg_rel, Wg_root,
      bg_rel.reshape(1, 1))


def _tc3_body(gp_ref, g2b_ref, batch_ref, z_ref, wc_ref, bc_ref, ls_ref,
              y_ref, wstd_ref):
    ones = jnp.ones((2, 1), jnp.float32)
    gate = lax.dot_general(gp_ref[...], ones, (((0,), (0,)), ((), ())),
                           preferred_element_type=jnp.float32) + g2b_ref[...]
    mask = (batch_ref[...] ==
            lax.broadcasted_iota(jnp.int32, (N_NODES, NBATCH), 1)
            ).astype(jnp.float32)
    m = jnp.max(jnp.where(mask > 0, gate, -3e38), axis=0, keepdims=True)
    m_n = jnp.sum(mask * m, axis=1, keepdims=True)
    g = jnp.exp(gate - m_n)
    ssum = jnp.sum(mask * g, axis=0, keepdims=True)
    s_n = jnp.sum(mask * ssum, axis=1, keepdims=True)
    gsm = g / (s_n + 1e-16)
    mg = mask * gsm
    pooled = lax.dot_general(mg, z_ref[...], (((0,), (0,)), ((), ())),
                             preferred_element_type=jnp.float32)
    logits = lax.dot_general(pooled, wc_ref[...], (((1,), (1,)), ((), ())),
                             preferred_element_type=jnp.float32) + bc_ref[...]
    mx = jnp.max(logits, axis=1, keepdims=True)
    ex = jnp.exp(logits - mx)
    y_ref[...] = ex / jnp.sum(ex, axis=1, keepdims=True)
    wstd_ref[...] = jnp.exp(ls_ref[...])


def _tc3(gatep, g2b, batch2d, z, Wc, bc, log_std):
    return pl.pallas_call(
        _tc3_body,
        out_shape=[
            jax.ShapeDtypeStruct((NBATCH, 2), jnp.float32),
            jax.ShapeDtypeStruct((1, 1), jnp.float32),
        ],
    )(gatep, g2b, batch2d, z, Wc, bc.reshape(1, -1), log_std.reshape(1, 1))


def kernel(x, edge_index, edge_weight, batch, W1_rel, b1_rel, W1_root,
           W2_rel, b2_rel, W2_root, Ws, bs, Wp, bp, Wg_rel, bg_rel,
           Wg_root, Wc, bc, log_std):
    src = edge_index[0]
    dst = edge_index[1]

    p1 = _edge_segsum(x, edge_index, edge_weight)
    h2, hroot, z_std = _tc1(p1, x, W1_rel, b1_rel, W1_root, W2_rel, b2_rel,
                            W2_root, Ws, bs)
    p2 = _edge_segsum(h2, edge_index, edge_weight)
    z, zn, g1, g2b = _tc2(p2, hroot, Wp, bp, Wg_rel, Wg_root, bg_rel)
    w_mu, gatep = _cos_gate(zn, g1.reshape(-1), src, dst, edge_weight)
    gatep2 = gatep.reshape(2, GPAD)[:, :N_NODES]
    y, w_std = _tc3(gatep2, g2b, batch.reshape(-1, 1), z, Wc, bc, log_std)

    return (y, w_mu, w_std.reshape(1), z, z, z_std)


# speedup vs baseline: 8.4990x; 1.2985x over previous
import functools
import jax
import jax.numpy as jnp
from jax import lax
from jax.experimental import pallas as pl
from jax.experimental.pallas import tpu as pltpu
from jax.experimental.pallas import tpu_sc as plsc

N_NODES = 10000
N_EDGES = 320000
NBATCH = 64
D = 128
CHUNK = 128
NCHUNK = N_EDGES // CHUNK
NWORK = 32
BASE_CNT = NCHUNK // NWORK
REM = NCHUNK - BASE_CNT * NWORK
ROWS_PT = 624
TAIL = N_NODES - 16 * ROWS_PT
TAIL_OFF = 16 * ROWS_PT
GPAD = 10240
GSH = GPAD // 16

_mesh = plsc.VectorSubcoreMesh(core_axis_name="c", subcore_axis_name="s")
_sc_params = pltpu.CompilerParams(needs_layout_passes=False)


def _worker_id():
    c = lax.axis_index("c")
    s = lax.axis_index("s")
    return c, s, c * 16 + s


def _scale_rows(rows_v, ew_v, lane):
    def scale16(i, _):
        ew16 = ew_v[pl.ds(i * 16, 16)]
        for l in range(16):
            e = i * 16 + l
            b = jnp.sum(jnp.where(lane == l, ew16, 0.0))
            for f in range(D // 16):
                rows_v[e, pl.ds(16 * f, 16)] = (
                    rows_v[e, pl.ds(16 * f, 16)] * b)
        return 0

    lax.fori_loop(0, CHUNK // 16, scale16, 0)


def _segsum_body(table_hbm, eidx_hbm, ew_hbm, out_hbm,
                 ei0, ei1, ew0, ew1, rows0, rows1, acc_sh,
                 gsem0, gsem1, ssem0, ssem1):
    c, s, w = _worker_id()
    eib = (ei0, ei1)
    ewb = (ew0, ew1)
    rows = (rows0, rows1)
    gsem = (gsem0, gsem1)
    ssem = (ssem0, ssem1)

    zv = jnp.zeros((16,), jnp.float32)

    def zrow(i, _):
        for f in range(D // 16):
            rows0[i, pl.ds(16 * f, 16)] = zv
        return 0

    lax.fori_loop(0, CHUNK, zrow, 0)
    for k, sz in enumerate((128, 128, 128, 128, 112)):
        pltpu.sync_copy(rows0.at[pl.ds(0, sz)],
                        acc_sh.at[pl.ds(s * ROWS_PT + k * 128, sz)])

    @pl.when(s == 0)
    def _():
        pltpu.sync_copy(rows0.at[pl.ds(0, TAIL)],
                        acc_sh.at[pl.ds(TAIL_OFF, TAIL)])

    plsc.subcore_barrier()

    cnt = BASE_CNT + jnp.where(w < REM, 1, 0)
    lane = lax.broadcasted_iota(jnp.int32, (16,), 0)

    def g_desc(b):
        return pltpu.make_async_copy(table_hbm.at[eib[b].at[0]], rows[b],
                                     gsem[b])

    def s_desc(b):
        return pltpu.make_async_copy(rows[b], acc_sh.at[eib[b].at[1]],
                                     ssem[b])

    def load_and_gather(k, b):
        base = (w + NWORK * k) * CHUNK
        pltpu.sync_copy(eidx_hbm.at[:, pl.ds(base, CHUNK)], eib[b])
        pltpu.sync_copy(ew_hbm.at[pl.ds(base, CHUNK)], ewb[b])
        g_desc(b).start()

    load_and_gather(0, 0)

    def pair_body(k2, _):
        for b in range(2):
            k = k2 * 2 + b

            @pl.when(k < cnt)
            def _():
                g_desc(b).wait()

                @pl.when(k + 1 < cnt)
                def _():
                    @pl.when(k >= 1)
                    def _():
                        s_desc(1 - b).wait()

                    load_and_gather(k + 1, 1 - b)

                _scale_rows(rows[b], ewb[b], lane)
                pltpu.async_copy(rows[b], acc_sh.at[eib[b].at[1]], ssem[b],
                                 add=True)

        return 0

    lax.fori_loop(0, (BASE_CNT + 2) // 2, pair_body, 0)
    s_desc(0).wait()
    s_desc(1).wait()
    plsc.subcore_barrier()
    pltpu.sync_copy(acc_sh.at[pl.ds(s * ROWS_PT, ROWS_PT)],
                    out_hbm.at[c, pl.ds(s * ROWS_PT, ROWS_PT)])

    @pl.when(s == 0)
    def _():
        pltpu.sync_copy(acc_sh.at[pl.ds(TAIL_OFF, TAIL)],
                        out_hbm.at[c, pl.ds(TAIL_OFF, TAIL)])


def _edge_segsum(table, eidx, ew):
    return pl.kernel(
        _segsum_body,
        out_type=jax.ShapeDtypeStruct((2, N_NODES, D), jnp.float32),
        mesh=_mesh,
        compiler_params=_sc_params,
        scratch_types=[
            pltpu.VMEM((2, CHUNK), jnp.int32),
            pltpu.VMEM((2, CHUNK), jnp.int32),
            pltpu.VMEM((CHUNK,), jnp.float32),
            pltpu.VMEM((CHUNK,), jnp.float32),
            pltpu.VMEM((CHUNK, D), jnp.float32),
            pltpu.VMEM((CHUNK, D), jnp.float32),
            pltpu.VMEM_SHARED((N_NODES, D), jnp.float32),
            pltpu.SemaphoreType.DMA,
            pltpu.SemaphoreType.DMA,
            pltpu.SemaphoreType.DMA,
            pltpu.SemaphoreType.DMA,
        ],
    )(table, eidx, ew)


def _cos_gate_body(zn_hbm, g1_hbm, eidx_hbm, ew_hbm,
                   wmu_hbm, gatep_hbm,
                   ei0, ei1, ew0, ew1, ra0, ra1, rb0, rb1, wm0, wm1,
                   wg0, wg1, g1v0, g1v1,
                   gacc_sh, sa0, sa1, sb0, sb1, sg0, sg1, sw0, sw1, ss0, ss1):
    c, s, w = _worker_id()
    eib = (ei0, ei1)
    ewb = (ew0, ew1)
    ra = (ra0, ra1)
    rb = (rb0, rb1)
    wm = (wm0, wm1)
    wg = (wg0, wg1)
    g1v = (g1v0, g1v1)
    sa = (sa0, sa1)
    sb = (sb0, sb1)
    sg = (sg0, sg1)
    sw = (sw0, sw1)
    ss = (ss0, ss1)

    def zwg(i, _):
        wg0[pl.ds(i * 16, 16)] = jnp.zeros((16,), jnp.float32)
        return 0

    lax.fori_loop(0, CHUNK // 16, zwg, 0)
    for k in range(GSH // CHUNK):
        pltpu.sync_copy(wg0, gacc_sh.at[pl.ds(s * GSH + k * CHUNK, CHUNK)])

    plsc.subcore_barrier()

    cnt = BASE_CNT + jnp.where(w < REM, 1, 0)
    lane = lax.broadcasted_iota(jnp.int32, (16,), 0)

    def ga_desc(b):
        return pltpu.make_async_copy(zn_hbm.at[eib[b].at[0]], ra[b], sa[b])

    def gb_desc(b):
        return pltpu.make_async_copy(zn_hbm.at[eib[b].at[1]], rb[b], sb[b])

    def gg_desc(b):
        return pltpu.make_async_copy(g1_hbm.at[eib[b].at[0]], g1v[b], sg[b])

    def gs_desc(b):
        return pltpu.make_async_copy(wg[b], gacc_sh.at[eib[b].at[1]], ss[b])

    def wm_desc(k, b):
        base = (w + NWORK * k) * CHUNK
        return pltpu.make_async_copy(wm[b], wmu_hbm.at[pl.ds(base, CHUNK)],
                                     sw[b])

    def load_and_gather(k, b):
        base = (w + NWORK * k) * CHUNK
        pltpu.sync_copy(eidx_hbm.at[:, pl.ds(base, CHUNK)], eib[b])
        pltpu.sync_copy(ew_hbm.at[pl.ds(base, CHUNK)], ewb[b])
        ga_desc(b).start()
        gb_desc(b).start()
        gg_desc(b).start()

    load_and_gather(0, 0)

    def pair_body(k2, _):
        for b in range(2):
            k = k2 * 2 + b

            @pl.when(k < cnt)
            def _():
                gg_desc(b).wait()

                @pl.when(k + 1 < cnt)
                def _():
                    @pl.when(k >= 1)
                    def _():
                        gs_desc(1 - b).wait()
                        wm_desc(k - 1, 1 - b).wait()

                    load_and_gather(k + 1, 1 - b)

                def gate16(i, _):
                    wv = ewb[b][pl.ds(i * 16, 16)]
                    wg[b][pl.ds(i * 16, 16)] = (
                        g1v[b][pl.ds(i * 16, 16)] * wv)
                    return 0

                lax.fori_loop(0, CHUNK // 16, gate16, 0)
                pltpu.async_copy(wg[b], gacc_sh.at[eib[b].at[1]], ss[b],
                                 add=True)

                ga_desc(b).wait()
                gb_desc(b).wait()

                def dot16(i, _):
                    out16 = jnp.zeros((16,), jnp.float32)
                    for l in range(16):
                        e = i * 16 + l
                        acc = ra[b][e, pl.ds(0, 16)] * rb[b][e, pl.ds(0, 16)]
                        for f in range(1, D // 16):
                            acc = acc + (ra[b][e, pl.ds(16 * f, 16)] *
                                         rb[b][e, pl.ds(16 * f, 16)])
                        out16 = jnp.where(lane == l, jnp.sum(acc), out16)
                    wm[b][pl.ds(i * 16, 16)] = out16
                    return 0

                lax.fori_loop(0, CHUNK // 16, dot16, 0)
                wm_desc(k, b).start()

        return 0

    lax.fori_loop(0, (BASE_CNT + 2) // 2, pair_body, 0)
    gs_desc(0).wait()
    gs_desc(1).wait()
    wm_desc(0, 0).wait()
    wm_desc(0, 1).wait()
    plsc.subcore_barrier()
    pltpu.sync_copy(gacc_sh.at[pl.ds(s * GSH, GSH)],
                    gatep_hbm.at[c, 0, pl.ds(s * GSH, GSH)])


def _cos_gate(zn, g1, eidx, ew):
    return pl.kernel(
        _cos_gate_body,
        out_type=(
            jax.ShapeDtypeStruct((N_EDGES,), jnp.float32),
            jax.ShapeDtypeStruct((2, 1, GPAD), jnp.float32),
        ),
        mesh=_mesh,
        compiler_params=_sc_params,
        scratch_types=[
            pltpu.VMEM((2, CHUNK), jnp.int32),
            pltpu.VMEM((2, CHUNK), jnp.int32),
            pltpu.VMEM((CHUNK,), jnp.float32),
            pltpu.VMEM((CHUNK,), jnp.float32),
            pltpu.VMEM((CHUNK, D), jnp.float32),
            pltpu.VMEM((CHUNK, D), jnp.float32),
            pltpu.VMEM((CHUNK, D), jnp.float32),
            pltpu.VMEM((CHUNK, D), jnp.float32),
            pltpu.VMEM((CHUNK,), jnp.float32),
            pltpu.VMEM((CHUNK,), jnp.float32),
            pltpu.VMEM((CHUNK,), jnp.float32),
            pltpu.VMEM((CHUNK,), jnp.float32),
            pltpu.VMEM((CHUNK,), jnp.float32),
            pltpu.VMEM((CHUNK,), jnp.float32),
            pltpu.VMEM_SHARED((GPAD,), jnp.float32),
            pltpu.SemaphoreType.DMA,
            pltpu.SemaphoreType.DMA,
            pltpu.SemaphoreType.DMA,
            pltpu.SemaphoreType.DMA,
            pltpu.SemaphoreType.DMA,
            pltpu.SemaphoreType.DMA,
            pltpu.SemaphoreType.DMA,
            pltpu.SemaphoreType.DMA,
            pltpu.SemaphoreType.DMA,
            pltpu.SemaphoreType.DMA,
        ],
    )(zn, g1, eidx, ew)


RB = 2000


def _tc1_body(p_ref, x_ref, w1r_ref, b1_ref, w1o_ref, w2r_ref, b2_ref,
              w2o_ref, ws_ref, bs_ref, h2_ref, hroot_ref, zstd_ref):
    agg = p_ref[0] + p_ref[1]
    dg = lambda a, w: lax.dot_general(a, w, (((1,), (1,)), ((), ())),
                                      preferred_element_type=jnp.float32)
    h = jax.nn.relu(dg(agg, w1r_ref[...]) + b1_ref[...] +
                    dg(x_ref[...], w1o_ref[...]))
    h2_ref[...] = dg(h, w2r_ref[...])
    hroot_ref[...] = dg(h, w2o_ref[...]) + b2_ref[...]
    zstd_ref[...] = jnp.exp(jnp.tanh(dg(h, ws_ref[...]) + bs_ref[...]))


def _tc1(p, x, W1_rel, b1, W1_root, W2_rel, b2, W2_root, Ws, bs):
    nb = N_NODES // RB
    full = lambda shape: pl.BlockSpec(shape, lambda i: (0,) * len(shape))
    return pl.pallas_call(
        _tc1_body,
        grid=(nb,),
        in_specs=[
            pl.BlockSpec((2, RB, D), lambda i: (0, i, 0)),
            pl.BlockSpec((RB, D), lambda i: (i, 0)),
            full((256, 128)), full((1, 256)), full((256, 128)),
            full((128, 256)), full((1, 128)), full((128, 256)),
            full((128, 256)), full((1, 128)),
        ],
        out_specs=[
            pl.BlockSpec((RB, D), lambda i: (i, 0)),
            pl.BlockSpec((RB, D), lambda i: (i, 0)),
            pl.BlockSpec((RB, D), lambda i: (i, 0)),
        ],
        out_shape=[
            jax.ShapeDtypeStruct((N_NODES, D), jnp.float32),
            jax.ShapeDtypeStruct((N_NODES, D), jnp.float32),
            jax.ShapeDtypeStruct((N_NODES, D), jnp.float32),
        ],
    )(p, x, W1_rel, b1.reshape(1, -1), W1_root, W2_rel, b2.reshape(1, -1),
      W2_root, Ws, bs.reshape(1, -1))


def _tc2_body(p_ref, hroot_ref, wp_ref, bp_ref, wgr_ref, wgo_ref, bg_ref,
              z_ref, zn_ref, g1_ref, g2b_ref):
    z = jnp.tanh(p_ref[0] + p_ref[1] + hroot_ref[...])
    z_ref[...] = z
    na = jnp.maximum(jnp.sqrt(jnp.sum(z * z, axis=1, keepdims=True)), 1e-8)
    zn_ref[...] = z / na
    x1 = lax.dot_general(z, wp_ref[...], (((1,), (1,)), ((), ())),
                         preferred_element_type=jnp.float32) + bp_ref[...]
    g1_ref[...] = jnp.sum(x1 * wgr_ref[...], axis=1, keepdims=True)
    g2b_ref[...] = (jnp.sum(x1 * wgo_ref[...], axis=1, keepdims=True) +
                    bg_ref[...])


def _tc2(p, hroot, Wp, bp, Wg_rel, Wg_root, bg_rel):
    nb = N_NODES // RB
    full = lambda shape: pl.BlockSpec(shape, lambda i: (0,) * len(shape))
    return pl.pallas_call(
        _tc2_body,
        grid=(nb,),
        in_specs=[
            pl.BlockSpec((2, RB, D), lambda i: (0, i, 0)),
            pl.BlockSpec((RB, D), lambda i: (i, 0)),
            full((128, 128)), full((1, 128)), full((1, 128)),
            full((1, 128)), full((1, 1)),
        ],
        out_specs=[
            pl.BlockSpec((RB, D), lambda i: (i, 0)),
            pl.BlockSpec((RB, D), lambda i: (i, 0)),
            pl.BlockSpec((RB, 1), lambda i: (i, 0)),
            pl.BlockSpec((RB, 1), lambda i: (i, 0)),
        ],
        out_shape=[
            jax.ShapeDtypeStruct((N_NODES, D), jnp.float32),
            jax.ShapeDtypeStruct((N_NODES, D), jnp.float32),
            jax.ShapeDtypeStruct((N_NODES, 1), jnp.float32),
            jax.ShapeDtypeStruct((N_NODES, 1), jnp.float32),
        ],
    )(p, hroot, Wp, bp.reshape(1, -1), Wg_rel, Wg_root,
      bg_rel.reshape(1, 1))


def _tc3_body(gp_ref, g2b_ref, batch_ref, z_ref, wc_ref, bc_ref, ls_ref,
              y_ref, wstd_ref):
    ones = jnp.ones((2, 1), jnp.float32)
    gate = lax.dot_general(gp_ref[...], ones, (((0,), (0,)), ((), ())),
                           preferred_element_type=jnp.float32) + g2b_ref[...]
    mask = (batch_ref[...] ==
            lax.broadcasted_iota(jnp.int32, (N_NODES, NBATCH), 1)
            ).astype(jnp.float32)
    m = jnp.max(jnp.where(mask > 0, gate, -3e38), axis=0, keepdims=True)
    m_n = jnp.sum(mask * m, axis=1, keepdims=True)
    g = jnp.exp(gate - m_n)
    ssum = jnp.sum(mask * g, axis=0, keepdims=True)
    s_n = jnp.sum(mask * ssum, axis=1, keepdims=True)
    gsm = g / (s_n + 1e-16)
    mg = mask * gsm
    pooled = lax.dot_general(mg, z_ref[...], (((0,), (0,)), ((), ())),
                             preferred_element_type=jnp.float32)
    logits = lax.dot_general(pooled, wc_ref[...], (((1,), (1,)), ((), ())),
                             preferred_element_type=jnp.float32) + bc_ref[...]
    mx = jnp.max(logits, axis=1, keepdims=True)
    ex = jnp.exp(logits - mx)
    y_ref[...] = ex / jnp.sum(ex, axis=1, keepdims=True)
    wstd_ref[...] = jnp.exp(ls_ref[...])


def _tc3(gatep, g2b, batch2d, z, Wc, bc, log_std):
    return pl.pallas_call(
        _tc3_body,
        out_shape=[
            jax.ShapeDtypeStruct((NBATCH, 2), jnp.float32),
            jax.ShapeDtypeStruct((1, 1), jnp.float32),
        ],
    )(gatep, g2b, batch2d, z, Wc, bc.reshape(1, -1), log_std.reshape(1, 1))


def kernel(x, edge_index, edge_weight, batch, W1_rel, b1_rel, W1_root,
           W2_rel, b2_rel, W2_root, Ws, bs, Wp, bp, Wg_rel, bg_rel,
           Wg_root, Wc, bc, log_std):
    src = edge_index[0]
    dst = edge_index[1]

    p1 = _edge_segsum(x, edge_index, edge_weight)
    h2, hroot, z_std = _tc1(p1, x, W1_rel, b1_rel, W1_root, W2_rel, b2_rel,
                            W2_root, Ws, bs)
    p2 = _edge_segsum(h2, edge_index, edge_weight)
    z, zn, g1, g2b = _tc2(p2, hroot, Wp, bp, Wg_rel, Wg_root, bg_rel)
    w_mu, gatep = _cos_gate(zn, g1.reshape(-1), edge_index, edge_weight)
    gatep2 = gatep.reshape(2, GPAD)[:, :N_NODES]
    y, w_std = _tc3(gatep2, g2b, batch.reshape(-1, 1), z, Wc, bc, log_std)

    return (y, w_mu, w_std.reshape(1), z, z, z_std)


# speedup vs baseline: 8.5694x; 1.0083x over previous
import functools
import jax
import jax.numpy as jnp
from jax import lax
from jax.experimental import pallas as pl
from jax.experimental.pallas import tpu as pltpu
from jax.experimental.pallas import tpu_sc as plsc

N_NODES = 10000
N_EDGES = 320000
NBATCH = 64
D = 128
CHUNK = 128
NCHUNK = N_EDGES // CHUNK
NWORK = 32
BASE_CNT = NCHUNK // NWORK
REM = NCHUNK - BASE_CNT * NWORK
ROWS_PT = 624
TAIL = N_NODES - 16 * ROWS_PT
TAIL_OFF = 16 * ROWS_PT
GPAD = 10240
GSH = GPAD // 16

_mesh = plsc.VectorSubcoreMesh(core_axis_name="c", subcore_axis_name="s")
_sc_params = pltpu.CompilerParams(needs_layout_passes=False)


def _worker_id():
    c = lax.axis_index("c")
    s = lax.axis_index("s")
    return c, s, c * 16 + s


def _scale_rows(rows_v, ew_v, lane):
    def scale16(i, _):
        ew16 = ew_v[pl.ds(i * 16, 16)]
        for l in range(16):
            e = i * 16 + l
            b = jnp.sum(jnp.where(lane == l, ew16, 0.0))
            for f in range(D // 16):
                rows_v[e, pl.ds(16 * f, 16)] = (
                    rows_v[e, pl.ds(16 * f, 16)] * b)
        return 0

    lax.fori_loop(0, CHUNK // 16, scale16, 0)


def _segsum_body(table_hbm, eidx_hbm, ew_hbm, out_hbm,
                 ei0, ei1, ei2, ew0, ew1, ew2, rows0, rows1, rows2, acc_sh,
                 gsem0, gsem1, gsem2, ssem0, ssem1, ssem2):
    c, s, w = _worker_id()
    eib = (ei0, ei1, ei2)
    ewb = (ew0, ew1, ew2)
    rows = (rows0, rows1, rows2)
    gsem = (gsem0, gsem1, gsem2)
    ssem = (ssem0, ssem1, ssem2)

    zv = jnp.zeros((16,), jnp.float32)

    def zrow(i, _):
        for f in range(D // 16):
            rows0[i, pl.ds(16 * f, 16)] = zv
        return 0

    lax.fori_loop(0, CHUNK, zrow, 0)
    for k, sz in enumerate((128, 128, 128, 128, 112)):
        pltpu.sync_copy(rows0.at[pl.ds(0, sz)],
                        acc_sh.at[pl.ds(s * ROWS_PT + k * 128, sz)])

    @pl.when(s == 0)
    def _():
        pltpu.sync_copy(rows0.at[pl.ds(0, TAIL)],
                        acc_sh.at[pl.ds(TAIL_OFF, TAIL)])

    plsc.subcore_barrier()

    cnt = BASE_CNT + jnp.where(w < REM, 1, 0)
    lane = lax.broadcasted_iota(jnp.int32, (16,), 0)

    def g_desc(b):
        return pltpu.make_async_copy(table_hbm.at[eib[b].at[0]], rows[b],
                                     gsem[b])

    def s_desc(b):
        return pltpu.make_async_copy(rows[b], acc_sh.at[eib[b].at[1]],
                                     ssem[b])

    def load_and_gather(k, b):
        base = (w + NWORK * k) * CHUNK
        pltpu.sync_copy(eidx_hbm.at[:, pl.ds(base, CHUNK)], eib[b])
        pltpu.sync_copy(ew_hbm.at[pl.ds(base, CHUNK)], ewb[b])
        g_desc(b).start()

    load_and_gather(0, 0)
    load_and_gather(1, 1)

    def ring_body(k3, _):
        for b in range(3):
            k = k3 * 3 + b

            @pl.when(k < cnt)
            def _():
                g_desc(b).wait()

                @pl.when(k >= 1)
                def _():
                    s_desc((b + 2) % 3).wait()

                @pl.when(k + 2 < cnt)
                def _():
                    load_and_gather(k + 2, (b + 2) % 3)

                _scale_rows(rows[b], ewb[b], lane)
                pltpu.async_copy(rows[b], acc_sh.at[eib[b].at[1]], ssem[b],
                                 add=True)

        return 0

    lax.fori_loop(0, (BASE_CNT + 1 + 2) // 3, ring_body, 0)
    @pl.when(w < REM)
    def _():
        s_desc((BASE_CNT + 1 - 1) % 3).wait()

    @pl.when(w >= REM)
    def _():
        s_desc((BASE_CNT - 1) % 3).wait()
    plsc.subcore_barrier()
    pltpu.sync_copy(acc_sh.at[pl.ds(s * ROWS_PT, ROWS_PT)],
                    out_hbm.at[c, pl.ds(s * ROWS_PT, ROWS_PT)])

    @pl.when(s == 0)
    def _():
        pltpu.sync_copy(acc_sh.at[pl.ds(TAIL_OFF, TAIL)],
                        out_hbm.at[c, pl.ds(TAIL_OFF, TAIL)])


def _edge_segsum(table, eidx, ew):
    return pl.kernel(
        _segsum_body,
        out_type=jax.ShapeDtypeStruct((2, N_NODES, D), jnp.float32),
        mesh=_mesh,
        compiler_params=_sc_params,
        scratch_types=[
            pltpu.VMEM((2, CHUNK), jnp.int32),
            pltpu.VMEM((2, CHUNK), jnp.int32),
            pltpu.VMEM((2, CHUNK), jnp.int32),
            pltpu.VMEM((CHUNK,), jnp.float32),
            pltpu.VMEM((CHUNK,), jnp.float32),
            pltpu.VMEM((CHUNK,), jnp.float32),
            pltpu.VMEM((CHUNK, D), jnp.float32),
            pltpu.VMEM((CHUNK, D), jnp.float32),
            pltpu.VMEM((CHUNK, D), jnp.float32),
            pltpu.VMEM_SHARED((N_NODES, D), jnp.float32),
            pltpu.SemaphoreType.DMA,
            pltpu.SemaphoreType.DMA,
            pltpu.SemaphoreType.DMA,
            pltpu.SemaphoreType.DMA,
            pltpu.SemaphoreType.DMA,
            pltpu.SemaphoreType.DMA,
        ],
    )(table, eidx, ew)


def _cos_gate_body(zn_hbm, g1_hbm, eidx_hbm, ew_hbm,
                   wmu_hbm, gatep_hbm,
                   ei0, ei1, ei2, ew0, ew1, ew2, ra0, ra1, ra2,
                   rb0, rb1, rb2, wm0, wm1, wm2,
                   wg0, wg1, wg2, g1v0, g1v1, g1v2,
                   gacc_sh, sa0, sa1, sa2, sb0, sb1, sb2, sg0, sg1, sg2,
                   sw0, sw1, sw2, ss0, ss1, ss2):
    c, s, w = _worker_id()
    eib = (ei0, ei1, ei2)
    ewb = (ew0, ew1, ew2)
    ra = (ra0, ra1, ra2)
    rb = (rb0, rb1, rb2)
    wm = (wm0, wm1, wm2)
    wg = (wg0, wg1, wg2)
    g1v = (g1v0, g1v1, g1v2)
    sa = (sa0, sa1, sa2)
    sb = (sb0, sb1, sb2)
    sg = (sg0, sg1, sg2)
    sw = (sw0, sw1, sw2)
    ss = (ss0, ss1, ss2)

    def zwg(i, _):
        wg0[pl.ds(i * 16, 16)] = jnp.zeros((16,), jnp.float32)
        return 0

    lax.fori_loop(0, CHUNK // 16, zwg, 0)
    for k in range(GSH // CHUNK):
        pltpu.sync_copy(wg0, gacc_sh.at[pl.ds(s * GSH + k * CHUNK, CHUNK)])

    plsc.subcore_barrier()

    cnt = BASE_CNT + jnp.where(w < REM, 1, 0)
    lane = lax.broadcasted_iota(jnp.int32, (16,), 0)

    def ga_desc(b):
        return pltpu.make_async_copy(zn_hbm.at[eib[b].at[0]], ra[b], sa[b])

    def gb_desc(b):
        return pltpu.make_async_copy(zn_hbm.at[eib[b].at[1]], rb[b], sb[b])

    def gg_desc(b):
        return pltpu.make_async_copy(g1_hbm.at[eib[b].at[0]], g1v[b], sg[b])

    def gs_desc(b):
        return pltpu.make_async_copy(wg[b], gacc_sh.at[eib[b].at[1]], ss[b])

    def wm_desc(k, b):
        base = (w + NWORK * k) * CHUNK
        return pltpu.make_async_copy(wm[b], wmu_hbm.at[pl.ds(base, CHUNK)],
                                     sw[b])

    def load_and_gather(k, b):
        base = (w + NWORK * k) * CHUNK
        pltpu.sync_copy(eidx_hbm.at[:, pl.ds(base, CHUNK)], eib[b])
        pltpu.sync_copy(ew_hbm.at[pl.ds(base, CHUNK)], ewb[b])
        ga_desc(b).start()
        gb_desc(b).start()
        gg_desc(b).start()

    load_and_gather(0, 0)
    load_and_gather(1, 1)

    def ring_body(k3, _):
        for b in range(3):
            k = k3 * 3 + b

            @pl.when(k < cnt)
            def _():
                gg_desc(b).wait()

                @pl.when(k >= 1)
                def _():
                    gs_desc((b + 2) % 3).wait()
                    wm_desc(0, (b + 2) % 3).wait()

                @pl.when(k + 2 < cnt)
                def _():
                    load_and_gather(k + 2, (b + 2) % 3)

                def gate16(i, _):
                    wv = ewb[b][pl.ds(i * 16, 16)]
                    wg[b][pl.ds(i * 16, 16)] = (
                        g1v[b][pl.ds(i * 16, 16)] * wv)
                    return 0

                lax.fori_loop(0, CHUNK // 16, gate16, 0)
                pltpu.async_copy(wg[b], gacc_sh.at[eib[b].at[1]], ss[b],
                                 add=True)

                ga_desc(b).wait()
                gb_desc(b).wait()

                def dot16(i, _):
                    out16 = jnp.zeros((16,), jnp.float32)
                    for l in range(16):
                        e = i * 16 + l
                        acc = ra[b][e, pl.ds(0, 16)] * rb[b][e, pl.ds(0, 16)]
                        for f in range(1, D // 16):
                            acc = acc + (ra[b][e, pl.ds(16 * f, 16)] *
                                         rb[b][e, pl.ds(16 * f, 16)])
                        out16 = jnp.where(lane == l, jnp.sum(acc), out16)
                    wm[b][pl.ds(i * 16, 16)] = out16
                    return 0

                lax.fori_loop(0, CHUNK // 16, dot16, 0)
                wm_desc(k, b).start()

        return 0

    lax.fori_loop(0, (BASE_CNT + 1 + 2) // 3, ring_body, 0)
    @pl.when(w < REM)
    def _():
        gs_desc((BASE_CNT + 1 - 1) % 3).wait()
        wm_desc(0, (BASE_CNT + 1 - 1) % 3).wait()

    @pl.when(w >= REM)
    def _():
        gs_desc((BASE_CNT - 1) % 3).wait()
        wm_desc(0, (BASE_CNT - 1) % 3).wait()
    plsc.subcore_barrier()
    pltpu.sync_copy(gacc_sh.at[pl.ds(s * GSH, GSH)],
                    gatep_hbm.at[c, 0, pl.ds(s * GSH, GSH)])


def _cos_gate(zn, g1, eidx, ew):
    return pl.kernel(
        _cos_gate_body,
        out_type=(
            jax.ShapeDtypeStruct((N_EDGES,), jnp.float32),
            jax.ShapeDtypeStruct((2, 1, GPAD), jnp.float32),
        ),
        mesh=_mesh,
        compiler_params=_sc_params,
        scratch_types=(
            [pltpu.VMEM((2, CHUNK), jnp.int32)] * 3
            + [pltpu.VMEM((CHUNK,), jnp.float32)] * 3
            + [pltpu.VMEM((CHUNK, D), jnp.float32)] * 6
            + [pltpu.VMEM((CHUNK,), jnp.float32)] * 9
            + [pltpu.VMEM_SHARED((GPAD,), jnp.float32)]
            + [pltpu.SemaphoreType.DMA] * 15
        ),
    )(zn, g1, eidx, ew)


RB = 2000


def _tc1_body(p_ref, x_ref, w1r_ref, b1_ref, w1o_ref, w2r_ref, b2_ref,
              w2o_ref, ws_ref, bs_ref, h2_ref, hroot_ref, zstd_ref):
    agg = p_ref[0] + p_ref[1]
    dg = lambda a, w: lax.dot_general(a, w, (((1,), (1,)), ((), ())),
                                      preferred_element_type=jnp.float32)
    h = jax.nn.relu(dg(agg, w1r_ref[...]) + b1_ref[...] +
                    dg(x_ref[...], w1o_ref[...]))
    h2_ref[...] = dg(h, w2r_ref[...])
    hroot_ref[...] = dg(h, w2o_ref[...]) + b2_ref[...]
    zstd_ref[...] = jnp.exp(jnp.tanh(dg(h, ws_ref[...]) + bs_ref[...]))


def _tc1(p, x, W1_rel, b1, W1_root, W2_rel, b2, W2_root, Ws, bs):
    nb = N_NODES // RB
    full = lambda shape: pl.BlockSpec(shape, lambda i: (0,) * len(shape))
    return pl.pallas_call(
        _tc1_body,
        grid=(nb,),
        in_specs=[
            pl.BlockSpec((2, RB, D), lambda i: (0, i, 0)),
            pl.BlockSpec((RB, D), lambda i: (i, 0)),
            full((256, 128)), full((1, 256)), full((256, 128)),
            full((128, 256)), full((1, 128)), full((128, 256)),
            full((128, 256)), full((1, 128)),
        ],
        out_specs=[
            pl.BlockSpec((RB, D), lambda i: (i, 0)),
            pl.BlockSpec((RB, D), lambda i: (i, 0)),
            pl.BlockSpec((RB, D), lambda i: (i, 0)),
        ],
        out_shape=[
            jax.ShapeDtypeStruct((N_NODES, D), jnp.float32),
            jax.ShapeDtypeStruct((N_NODES, D), jnp.float32),
            jax.ShapeDtypeStruct((N_NODES, D), jnp.float32),
        ],
    )(p, x, W1_rel, b1.reshape(1, -1), W1_root, W2_rel, b2.reshape(1, -1),
      W2_root, Ws, bs.reshape(1, -1))


def _tc2_body(p_ref, hroot_ref, wp_ref, bp_ref, wgr_ref, wgo_ref, bg_ref,
              z_ref, zn_ref, g1_ref, g2b_ref):
    z = jnp.tanh(p_ref[0] + p_ref[1] + hroot_ref[...])
    z_ref[...] = z
    na = jnp.maximum(jnp.sqrt(jnp.sum(z * z, axis=1, keepdims=True)), 1e-8)
    zn_ref[...] = z / na
    x1 = lax.dot_general(z, wp_ref[...], (((1,), (1,)), ((), ())),
                         preferred_element_type=jnp.float32) + bp_ref[...]
    g1_ref[...] = jnp.sum(x1 * wgr_ref[...], axis=1, keepdims=True)
    g2b_ref[...] = (jnp.sum(x1 * wgo_ref[...], axis=1, keepdims=True) +
                    bg_ref[...])


def _tc2(p, hroot, Wp, bp, Wg_rel, Wg_root, bg_rel):
    nb = N_NODES // RB
    full = lambda shape: pl.BlockSpec(shape, lambda i: (0,) * len(shape))
    return pl.pallas_call(
        _tc2_body,
        grid=(nb,),
        in_specs=[
            pl.BlockSpec((2, RB, D), lambda i: (0, i, 0)),
            pl.BlockSpec((RB, D), lambda i: (i, 0)),
            full((128, 128)), full((1, 128)), full((1, 128)),
            full((1, 128)), full((1, 1)),
        ],
        out_specs=[
            pl.BlockSpec((RB, D), lambda i: (i, 0)),
            pl.BlockSpec((RB, D), lambda i: (i, 0)),
            pl.BlockSpec((RB, 1), lambda i: (i, 0)),
            pl.BlockSpec((RB, 1), lambda i: (i, 0)),
        ],
        out_shape=[
            jax.ShapeDtypeStruct((N_NODES, D), jnp.float32),
            jax.ShapeDtypeStruct((N_NODES, D), jnp.float32),
            jax.ShapeDtypeStruct((N_NODES, 1), jnp.float32),
            jax.ShapeDtypeStruct((N_NODES, 1), jnp.float32),
        ],
    )(p, hroot, Wp, bp.reshape(1, -1), Wg_rel, Wg_root,
      bg_rel.reshape(1, 1))


def _tc3_body(gp_ref, g2b_ref, batch_ref, z_ref, wc_ref, bc_ref, ls_ref,
              y_ref, wstd_ref):
    ones = jnp.ones((2, 1), jnp.float32)
    gate = lax.dot_general(gp_ref[...], ones, (((0,), (0,)), ((), ())),
                           preferred_element_type=jnp.float32) + g2b_ref[...]
    mask = (batch_ref[...] ==
            lax.broadcasted_iota(jnp.int32, (N_NODES, NBATCH), 1)
            ).astype(jnp.float32)
    m = jnp.max(jnp.where(mask > 0, gate, -3e38), axis=0, keepdims=True)
    m_n = jnp.sum(mask * m, axis=1, keepdims=True)
    g = jnp.exp(gate - m_n)
    ssum = jnp.sum(mask * g, axis=0, keepdims=True)
    s_n = jnp.sum(mask * ssum, axis=1, keepdims=True)
    gsm = g / (s_n + 1e-16)
    mg = mask * gsm
    pooled = lax.dot_general(mg, z_ref[...], (((0,), (0,)), ((), ())),
                             preferred_element_type=jnp.float32)
    logits = lax.dot_general(pooled, wc_ref[...], (((1,), (1,)), ((), ())),
                             preferred_element_type=jnp.float32) + bc_ref[...]
    mx = jnp.max(logits, axis=1, keepdims=True)
    ex = jnp.exp(logits - mx)
    y_ref[...] = ex / jnp.sum(ex, axis=1, keepdims=True)
    wstd_ref[...] = jnp.exp(ls_ref[...])


def _tc3(gatep, g2b, batch2d, z, Wc, bc, log_std):
    return pl.pallas_call(
        _tc3_body,
        out_shape=[
            jax.ShapeDtypeStruct((NBATCH, 2), jnp.float32),
            jax.ShapeDtypeStruct((1, 1), jnp.float32),
        ],
    )(gatep, g2b, batch2d, z, Wc, bc.reshape(1, -1), log_std.reshape(1, 1))


def kernel(x, edge_index, edge_weight, batch, W1_rel, b1_rel, W1_root,
           W2_rel, b2_rel, W2_root, Ws, bs, Wp, bp, Wg_rel, bg_rel,
           Wg_root, Wc, bc, log_std):
    src = edge_index[0]
    dst = edge_index[1]

    p1 = _edge_segsum(x, edge_index, edge_weight)
    h2, hroot, z_std = _tc1(p1, x, W1_rel, b1_rel, W1_root, W2_rel, b2_rel,
                            W2_root, Ws, bs)
    p2 = _edge_segsum(h2, edge_index, edge_weight)
    z, zn, g1, g2b = _tc2(p2, hroot, Wp, bp, Wg_rel, Wg_root, bg_rel)
    w_mu, gatep = _cos_gate(zn, g1.reshape(-1), edge_index, edge_weight)
    gatep2 = gatep.reshape(2, GPAD)[:, :N_NODES]
    y, w_std = _tc3(gatep2, g2b, batch.reshape(-1, 1), z, Wc, bc, log_std)

    return (y, w_mu, w_std.reshape(1), z, z, z_std)


# speedup vs baseline: 11.1730x; 1.3038x over previous
import functools
import jax
import jax.numpy as jnp
from jax import lax
from jax.experimental import pallas as pl
from jax.experimental.pallas import tpu as pltpu
from jax.experimental.pallas import tpu_sc as plsc

N_NODES = 10000
N_EDGES = 320000
NBATCH = 64
D = 128
CHUNK = 128
NCHUNK = N_EDGES // CHUNK
NWORK = 32
BASE_CNT = NCHUNK // NWORK
REM = NCHUNK - BASE_CNT * NWORK
ROWS_PT = 624
TAIL = N_NODES - 16 * ROWS_PT
TAIL_OFF = 16 * ROWS_PT
GPAD = 10240
GSH = GPAD // 16

_mesh = plsc.VectorSubcoreMesh(core_axis_name="c", subcore_axis_name="s")
_sc_params = pltpu.CompilerParams(needs_layout_passes=False)


def _worker_id():
    c = lax.axis_index("c")
    s = lax.axis_index("s")
    return c, s, c * 16 + s


def _scale_rows(rows_v, ew_v, lane):
    def scale16(i, _):
        ew16 = lax.bitcast_convert_type(ew_v[pl.ds(i * 16, 16)], jnp.float32)
        for l in range(16):
            e = i * 16 + l
            b = jnp.sum(jnp.where(lane == l, ew16, 0.0))
            for f in range(D // 16):
                rows_v[e, pl.ds(16 * f, 16)] = (
                    rows_v[e, pl.ds(16 * f, 16)] * b)
        return 0

    lax.fori_loop(0, CHUNK // 16, scale16, 0)


def _segsum_body(table_hbm, cat3_hbm, out_hbm,
                 ib0, ib1, ib2, ib3, rows0, rows1, acc_sh,
                 is0, is1, is2, is3, gsem0, gsem1, ssem0, ssem1):
    c, s, w = _worker_id()
    ib = (ib0, ib1, ib2, ib3)
    isem = (is0, is1, is2, is3)
    rows = (rows0, rows1)
    gsem = (gsem0, gsem1)
    ssem = (ssem0, ssem1)

    cnt = BASE_CNT + jnp.where(w < REM, 1, 0)
    lane = lax.broadcasted_iota(jnp.int32, (16,), 0)

    def i_desc(k, j):
        base = (w + NWORK * k) * CHUNK
        return pltpu.make_async_copy(cat3_hbm.at[:, pl.ds(base, CHUNK)],
                                     ib[j], isem[j])

    def g_desc(j, b):
        return pltpu.make_async_copy(table_hbm.at[ib[j].at[0]], rows[b],
                                     gsem[b])

    def s_desc(j, b):
        return pltpu.make_async_copy(rows[b], acc_sh.at[ib[j].at[1]],
                                     ssem[b])

    i_desc(0, 0).start()
    i_desc(1, 1).start()
    i_desc(2, 2).start()

    zv = jnp.zeros((16,), jnp.float32)

    def zrow(i, _):
        for f in range(D // 16):
            rows0[i, pl.ds(16 * f, 16)] = zv
        return 0

    lax.fori_loop(0, CHUNK, zrow, 0)
    for k, sz in enumerate((128, 128, 128, 128, 112)):
        pltpu.sync_copy(rows0.at[pl.ds(0, sz)],
                        acc_sh.at[pl.ds(s * ROWS_PT + k * 128, sz)])

    @pl.when(s == 0)
    def _():
        pltpu.sync_copy(rows0.at[pl.ds(0, TAIL)],
                        acc_sh.at[pl.ds(TAIL_OFF, TAIL)])

    plsc.subcore_barrier()

    i_desc(0, 0).wait()
    g_desc(0, 0).start()

    def ring_body(k4, _):
        for b4 in range(4):
            k = k4 * 4 + b4
            b = b4 % 2

            @pl.when(k < cnt)
            def _():
                g_desc(b4, b).wait()

                @pl.when(k >= 1)
                def _():
                    s_desc((b4 + 3) % 4, 1 - b).wait()

                @pl.when(k + 3 < cnt)
                def _():
                    i_desc(k + 3, (b4 + 3) % 4).start()

                @pl.when(k + 1 < cnt)
                def _():
                    i_desc(0, (b4 + 1) % 4).wait()
                    g_desc((b4 + 1) % 4, 1 - b).start()

                _scale_rows(rows[b], ib[b4].at[2], lane)
                pltpu.async_copy(rows[b], acc_sh.at[ib[b4].at[1]], ssem[b],
                                 add=True)

        return 0

    lax.fori_loop(0, (BASE_CNT + 1 + 3) // 4, ring_body, 0)
    @pl.when(w < REM)
    def _():
        s_desc((BASE_CNT + 1 - 1) % 4, (BASE_CNT + 1 - 1) % 2).wait()

    @pl.when(w >= REM)
    def _():
        s_desc((BASE_CNT - 1) % 4, (BASE_CNT - 1) % 2).wait()
    plsc.subcore_barrier()
    pltpu.sync_copy(acc_sh.at[pl.ds(s * ROWS_PT, ROWS_PT)],
                    out_hbm.at[c, pl.ds(s * ROWS_PT, ROWS_PT)])

    @pl.when(s == 0)
    def _():
        pltpu.sync_copy(acc_sh.at[pl.ds(TAIL_OFF, TAIL)],
                        out_hbm.at[c, pl.ds(TAIL_OFF, TAIL)])


def _edge_segsum(table, cat3):
    return pl.kernel(
        _segsum_body,
        out_type=jax.ShapeDtypeStruct((2, N_NODES, D), jnp.float32),
        mesh=_mesh,
        compiler_params=_sc_params,
        scratch_types=(
            [pltpu.VMEM((3, CHUNK), jnp.int32)] * 4
            + [pltpu.VMEM((CHUNK, D), jnp.float32)] * 2
            + [pltpu.VMEM_SHARED((N_NODES, D), jnp.float32)]
            + [pltpu.SemaphoreType.DMA] * 8
        ),
    )(table, cat3)


def _cos_gate_body(zn_hbm, g1_hbm, cat3_hbm,
                   wmu_hbm, gatep_hbm,
                   ib0, ib1, ib2, ra0, ra1, ra2,
                   rb0, rb1, rb2, wm0, wm1, wm2,
                   wg0, wg1, wg2, g1v0, g1v1, g1v2,
                   gacc_sh, is0, is1, is2, sa0, sa1, sa2, sb0, sb1, sb2,
                   sg0, sg1, sg2, sw0, sw1, sw2, ss0, ss1, ss2):
    c, s, w = _worker_id()
    ib = (ib0, ib1, ib2)
    isem = (is0, is1, is2)
    ra = (ra0, ra1, ra2)
    rb = (rb0, rb1, rb2)
    wm = (wm0, wm1, wm2)
    wg = (wg0, wg1, wg2)
    g1v = (g1v0, g1v1, g1v2)
    sa = (sa0, sa1, sa2)
    sb = (sb0, sb1, sb2)
    sg = (sg0, sg1, sg2)
    sw = (sw0, sw1, sw2)
    ss = (ss0, ss1, ss2)

    cnt = BASE_CNT + jnp.where(w < REM, 1, 0)
    lane = lax.broadcasted_iota(jnp.int32, (16,), 0)

    def i_desc(k, j):
        base = (w + NWORK * k) * CHUNK
        return pltpu.make_async_copy(cat3_hbm.at[:, pl.ds(base, CHUNK)],
                                     ib[j], isem[j])

    def ga_desc(b):
        return pltpu.make_async_copy(zn_hbm.at[ib[b].at[0]], ra[b], sa[b])

    def gb_desc(b):
        return pltpu.make_async_copy(zn_hbm.at[ib[b].at[1]], rb[b], sb[b])

    def gg_desc(b):
        return pltpu.make_async_copy(g1_hbm.at[ib[b].at[0]], g1v[b], sg[b])

    def gs_desc(b):
        return pltpu.make_async_copy(wg[b], gacc_sh.at[ib[b].at[1]], ss[b])

    def wm_desc(k, b):
        base = (w + NWORK * k) * CHUNK
        return pltpu.make_async_copy(wm[b], wmu_hbm.at[pl.ds(base, CHUNK)],
                                     sw[b])

    def start_gathers(b):
        ga_desc(b).start()
        gb_desc(b).start()
        gg_desc(b).start()

    i_desc(0, 0).start()
    i_desc(1, 1).start()

    def zwg(i, _):
        wg0[pl.ds(i * 16, 16)] = jnp.zeros((16,), jnp.float32)
        return 0

    lax.fori_loop(0, CHUNK // 16, zwg, 0)
    for k in range(GSH // CHUNK):
        pltpu.sync_copy(wg0, gacc_sh.at[pl.ds(s * GSH + k * CHUNK, CHUNK)])

    plsc.subcore_barrier()

    i_desc(0, 0).wait()
    start_gathers(0)

    def ring_body(k3, _):
        for b in range(3):
            k = k3 * 3 + b

            @pl.when(k < cnt)
            def _():
                gg_desc(b).wait()

                @pl.when(k >= 1)
                def _():
                    gs_desc((b + 2) % 3).wait()
                    wm_desc(0, (b + 2) % 3).wait()

                @pl.when(k + 2 < cnt)
                def _():
                    i_desc(k + 2, (b + 2) % 3).start()

                @pl.when(k + 1 < cnt)
                def _():
                    i_desc(0, (b + 1) % 3).wait()
                    start_gathers((b + 1) % 3)

                def gate16(i, _):
                    wv = lax.bitcast_convert_type(
                        ib[b][2, pl.ds(i * 16, 16)], jnp.float32)
                    wg[b][pl.ds(i * 16, 16)] = (
                        g1v[b][pl.ds(i * 16, 16)] * wv)
                    return 0

                lax.fori_loop(0, CHUNK // 16, gate16, 0)
                pltpu.async_copy(wg[b], gacc_sh.at[ib[b].at[1]], ss[b],
                                 add=True)

                ga_desc(b).wait()
                gb_desc(b).wait()

                def dot16(i, _):
                    out16 = jnp.zeros((16,), jnp.float32)
                    for l in range(16):
                        e = i * 16 + l
                        acc = ra[b][e, pl.ds(0, 16)] * rb[b][e, pl.ds(0, 16)]
                        for f in range(1, D // 16):
                            acc = acc + (ra[b][e, pl.ds(16 * f, 16)] *
                                         rb[b][e, pl.ds(16 * f, 16)])
                        out16 = jnp.where(lane == l, jnp.sum(acc), out16)
                    wm[b][pl.ds(i * 16, 16)] = out16
                    return 0

                lax.fori_loop(0, CHUNK // 16, dot16, 0)
                wm_desc(k, b).start()

        return 0

    lax.fori_loop(0, (BASE_CNT + 1 + 2) // 3, ring_body, 0)
    @pl.when(w < REM)
    def _():
        gs_desc((BASE_CNT + 1 - 1) % 3).wait()
        wm_desc(0, (BASE_CNT + 1 - 1) % 3).wait()

    @pl.when(w >= REM)
    def _():
        gs_desc((BASE_CNT - 1) % 3).wait()
        wm_desc(0, (BASE_CNT - 1) % 3).wait()
    plsc.subcore_barrier()
    pltpu.sync_copy(gacc_sh.at[pl.ds(s * GSH, GSH)],
                    gatep_hbm.at[c, 0, pl.ds(s * GSH, GSH)])


def _cos_gate(zn, g1, cat3):
    return pl.kernel(
        _cos_gate_body,
        out_type=(
            jax.ShapeDtypeStruct((N_EDGES,), jnp.float32),
            jax.ShapeDtypeStruct((2, 1, GPAD), jnp.float32),
        ),
        mesh=_mesh,
        compiler_params=_sc_params,
        scratch_types=(
            [pltpu.VMEM((3, CHUNK), jnp.int32)] * 3
            + [pltpu.VMEM((CHUNK, D), jnp.float32)] * 6
            + [pltpu.VMEM((CHUNK,), jnp.float32)] * 9
            + [pltpu.VMEM_SHARED((GPAD,), jnp.float32)]
            + [pltpu.SemaphoreType.DMA] * 18
        ),
    )(zn, g1, cat3)


RB = 2000


def _tc1_body(p_ref, x_ref, w1r_ref, b1_ref, w1o_ref, w2r_ref, b2_ref,
              w2o_ref, ws_ref, bs_ref, h2_ref, hroot_ref, zstd_ref):
    agg = p_ref[0] + p_ref[1]
    dg = lambda a, w: lax.dot_general(a, w, (((1,), (1,)), ((), ())),
                                      preferred_element_type=jnp.float32)
    h = jax.nn.relu(dg(agg, w1r_ref[...]) + b1_ref[...] +
                    dg(x_ref[...], w1o_ref[...]))
    h2_ref[...] = dg(h, w2r_ref[...])
    hroot_ref[...] = dg(h, w2o_ref[...]) + b2_ref[...]
    zstd_ref[...] = jnp.exp(jnp.tanh(dg(h, ws_ref[...]) + bs_ref[...]))


def _tc1(p, x, W1_rel, b1, W1_root, W2_rel, b2, W2_root, Ws, bs):
    nb = N_NODES // RB
    full = lambda shape: pl.BlockSpec(shape, lambda i: (0,) * len(shape))
    return pl.pallas_call(
        _tc1_body,
        grid=(nb,),
        in_specs=[
            pl.BlockSpec((2, RB, D), lambda i: (0, i, 0)),
            pl.BlockSpec((RB, D), lambda i: (i, 0)),
            full((256, 128)), full((1, 256)), full((256, 128)),
            full((128, 256)), full((1, 128)), full((128, 256)),
            full((128, 256)), full((1, 128)),
        ],
        out_specs=[
            pl.BlockSpec((RB, D), lambda i: (i, 0)),
            pl.BlockSpec((RB, D), lambda i: (i, 0)),
            pl.BlockSpec((RB, D), lambda i: (i, 0)),
        ],
        out_shape=[
            jax.ShapeDtypeStruct((N_NODES, D), jnp.float32),
            jax.ShapeDtypeStruct((N_NODES, D), jnp.float32),
            jax.ShapeDtypeStruct((N_NODES, D), jnp.float32),
        ],
    )(p, x, W1_rel, b1.reshape(1, -1), W1_root, W2_rel, b2.reshape(1, -1),
      W2_root, Ws, bs.reshape(1, -1))


def _tc2_body(p_ref, hroot_ref, wp_ref, bp_ref, wgr_ref, wgo_ref, bg_ref,
              z_ref, zn_ref, g1_ref, g2b_ref):
    z = jnp.tanh(p_ref[0] + p_ref[1] + hroot_ref[...])
    z_ref[...] = z
    na = jnp.maximum(jnp.sqrt(jnp.sum(z * z, axis=1, keepdims=True)), 1e-8)
    zn_ref[...] = z / na
    x1 = lax.dot_general(z, wp_ref[...], (((1,), (1,)), ((), ())),
                         preferred_element_type=jnp.float32) + bp_ref[...]
    g1_ref[...] = jnp.sum(x1 * wgr_ref[...], axis=1, keepdims=True)
    g2b_ref[...] = (jnp.sum(x1 * wgo_ref[...], axis=1, keepdims=True) +
                    bg_ref[...])


def _tc2(p, hroot, Wp, bp, Wg_rel, Wg_root, bg_rel):
    nb = N_NODES // RB
    full = lambda shape: pl.BlockSpec(shape, lambda i: (0,) * len(shape))
    return pl.pallas_call(
        _tc2_body,
        grid=(nb,),
        in_specs=[
            pl.BlockSpec((2, RB, D), lambda i: (0, i, 0)),
            pl.BlockSpec((RB, D), lambda i: (i, 0)),
            full((128, 128)), full((1, 128)), full((1, 128)),
            full((1, 128)), full((1, 1)),
        ],
        out_specs=[
            pl.BlockSpec((RB, D), lambda i: (i, 0)),
            pl.BlockSpec((RB, D), lambda i: (i, 0)),
            pl.BlockSpec((RB, 1), lambda i: (i, 0)),
            pl.BlockSpec((RB, 1), lambda i: (i, 0)),
        ],
        out_shape=[
            jax.ShapeDtypeStruct((N_NODES, D), jnp.float32),
            jax.ShapeDtypeStruct((N_NODES, D), jnp.float32),
            jax.ShapeDtypeStruct((N_NODES, 1), jnp.float32),
            jax.ShapeDtypeStruct((N_NODES, 1), jnp.float32),
        ],
    )(p, hroot, Wp, bp.reshape(1, -1), Wg_rel, Wg_root,
      bg_rel.reshape(1, 1))


def _tc3_body(gp_ref, g2b_ref, batch_ref, z_ref, wc_ref, bc_ref, ls_ref,
              y_ref, wstd_ref):
    ones = jnp.ones((2, 1), jnp.float32)
    gate = lax.dot_general(gp_ref[...], ones, (((0,), (0,)), ((), ())),
                           preferred_element_type=jnp.float32) + g2b_ref[...]
    mask = (batch_ref[...] ==
            lax.broadcasted_iota(jnp.int32, (N_NODES, NBATCH), 1)
            ).astype(jnp.float32)
    m = jnp.max(jnp.where(mask > 0, gate, -3e38), axis=0, keepdims=True)
    m_n = jnp.sum(mask * m, axis=1, keepdims=True)
    g = jnp.exp(gate - m_n)
    ssum = jnp.sum(mask * g, axis=0, keepdims=True)
    s_n = jnp.sum(mask * ssum, axis=1, keepdims=True)
    gsm = g / (s_n + 1e-16)
    mg = mask * gsm
    pooled = lax.dot_general(mg, z_ref[...], (((0,), (0,)), ((), ())),
                             preferred_element_type=jnp.float32)
    logits = lax.dot_general(pooled, wc_ref[...], (((1,), (1,)), ((), ())),
                             preferred_element_type=jnp.float32) + bc_ref[...]
    mx = jnp.max(logits, axis=1, keepdims=True)
    ex = jnp.exp(logits - mx)
    y_ref[...] = ex / jnp.sum(ex, axis=1, keepdims=True)
    wstd_ref[...] = jnp.exp(ls_ref[...])


def _tc3(gatep, g2b, batch2d, z, Wc, bc, log_std):
    return pl.pallas_call(
        _tc3_body,
        out_shape=[
            jax.ShapeDtypeStruct((NBATCH, 2), jnp.float32),
            jax.ShapeDtypeStruct((1, 1), jnp.float32),
        ],
    )(gatep, g2b, batch2d, z, Wc, bc.reshape(1, -1), log_std.reshape(1, 1))


def kernel(x, edge_index, edge_weight, batch, W1_rel, b1_rel, W1_root,
           W2_rel, b2_rel, W2_root, Ws, bs, Wp, bp, Wg_rel, bg_rel,
           Wg_root, Wc, bc, log_std):
    cat3 = jnp.concatenate(
        [edge_index,
         lax.bitcast_convert_type(edge_weight, jnp.int32)[None]], axis=0)

    p1 = _edge_segsum(x, cat3)
    h2, hroot, z_std = _tc1(p1, x, W1_rel, b1_rel, W1_root, W2_rel, b2_rel,
                            W2_root, Ws, bs)
    p2 = _edge_segsum(h2, cat3)
    z, zn, g1, g2b = _tc2(p2, hroot, Wp, bp, Wg_rel, Wg_root, bg_rel)
    w_mu, gatep = _cos_gate(zn, g1.reshape(-1), cat3)
    gatep2 = gatep.reshape(2, GPAD)[:, :N_NODES]
    y, w_std = _tc3(gatep2, g2b, batch.reshape(-1, 1), z, Wc, bc, log_std)

    return (y, w_mu, w_std.reshape(1), z, z, z_std)
